# baseline (speedup 1.0000x reference)
import functools

import jax
import jax.numpy as jnp
from jax import lax
from jax.experimental import pallas as pl
from jax.experimental.pallas import tpu as pltpu
from jax.experimental.pallas import tpu_sc as plsc

F32 = jnp.float32

NC = 2
NS = 16
NW = NC * NS
NPAD = 10240
RPS = NPAD // NS


@functools.cache
def _mesh():
    return plsc.VectorSubcoreMesh(
        core_axis_name="c", subcore_axis_name="s",
        num_cores=NC, num_subcores=NS,
    )


@functools.cache
def _make_gather2sum(n_rows, width, n_edges, subtract, dtype=F32):
    per = n_edges // NW
    assert per * NW == n_edges
    CH = 64
    n_full = per // CH
    tail = per - n_full * CH
    ngroups = n_full // 2
    assert n_full % 2 == 0 and tail % 8 == 0
    isbf = dtype == jnp.bfloat16
    wdt = jnp.int32 if isbf else F32
    nslice = width // 16

    scratch = [
        pltpu.VMEM((per,), jnp.int32),
        pltpu.VMEM((per,), jnp.int32),
    ]
    for _ in range(2):
        scratch += [pltpu.VMEM((CH, width), wdt)] * 3
    scratch += [pltpu.SemaphoreType.DMA] * 4
    if tail:
        scratch += [pltpu.VMEM((tail, width), wdt)] * 3

    @functools.partial(
        pl.kernel,
        out_type=jax.ShapeDtypeStruct((n_edges, width), wdt),
        mesh=_mesh(),
        scratch_types=scratch,
    )
    def gather2sum(ta, tb, ia, ib, out, idxa, idxb,
                   a0, b0, o0, a1, b1, o1, sa0, sb0, sa1, sb1, *tails):
        wid = lax.axis_index("s") * NC + lax.axis_index("c")
        base0 = wid * per
        pltpu.sync_copy(ia.at[pl.ds(base0, per)], idxa)
        pltpu.sync_copy(ib.at[pl.ds(base0, per)], idxb)
        pairs = ((a0, b0, o0, sa0, sb0), (a1, b1, o1, sa1, sb1))

        def issue(k, p):
            ba, bb, _, sa, sb = p
            pltpu.async_copy(ta.at[idxa.at[pl.ds(k * CH, CH)]], ba, sa)
            pltpu.async_copy(tb.at[idxb.at[pl.ds(k * CH, CH)]], bb, sb)

        def wait(k, p):
            ba, bb, _, sa, sb = p
            pltpu.make_async_copy(
                ta.at[idxa.at[pl.ds(k * CH, CH)]], ba, sa).wait()
            pltpu.make_async_copy(
                tb.at[idxb.at[pl.ds(k * CH, CH)]], bb, sb).wait()

        def combine(ba, bb, bo, n):
            def rowbody(r, c):
                for j in range(nslice):
                    sl = (r, pl.ds(j * 16, 16))
                    if isbf:
                        va = plsc.bitcast(ba[sl], jnp.bfloat16)
                        vb = plsc.bitcast(bb[sl], jnp.bfloat16)
                        v = va - vb if subtract else va + vb
                        bo[sl] = plsc.bitcast(v, jnp.int32)
                    else:
                        v = ba[sl] - bb[sl] if subtract else ba[sl] + bb[sl]
                        bo[sl] = v
                return c
            lax.fori_loop(0, n, rowbody, 0)

        for b in range(2):
            issue(b, pairs[b])

        def group(g, c):
            for b in range(2):
                k = 2 * g + b
                p = pairs[b]
                wait(k, p)
                combine(p[0], p[1], p[2], CH)
                issue(k + 2, p)
                pltpu.sync_copy(p[2], out.at[pl.ds(base0 + k * CH, CH)])
            return c

        lax.fori_loop(0, ngroups - 1, group, 0)
        for b in range(2):
            k = 2 * (ngroups - 1) + b
            p = pairs[b]
            wait(k, p)
            combine(p[0], p[1], p[2], CH)
            pltpu.sync_copy(p[2], out.at[pl.ds(base0 + k * CH, CH)])
        if tail:
            tba, tbb, tbo = tails
            kb = n_full * CH
            pltpu.async_copy(
                ta.at[idxa.at[pl.ds(kb, tail)]], tba, sa0).wait()
            pltpu.async_copy(
                tb.at[idxb.at[pl.ds(kb, tail)]], tbb, sb0).wait()
            combine(tba, tbb, tbo, tail)
            pltpu.sync_copy(tbo, out.at[pl.ds(base0 + kb, tail)])

    return gather2sum


@functools.cache
def _make_gather2(n_rows, width, n_edges, wdt):
    per = n_edges // NW
    assert per * NW == n_edges
    CH = 64
    n_full = per // CH
    tail = per - n_full * CH
    ngroups = n_full // 2
    assert n_full % 2 == 0 and tail % 8 == 0

    scratch = [
        pltpu.VMEM((per,), jnp.int32),
        pltpu.VMEM((per,), jnp.int32),
    ]
    for _ in range(2):
        scratch += [pltpu.VMEM((CH, width), wdt)] * 2
    scratch += [pltpu.SemaphoreType.DMA] * 4
    if tail:
        scratch += [pltpu.VMEM((tail, width), wdt)] * 2

    @functools.partial(
        pl.kernel,
        out_type=(
            jax.ShapeDtypeStruct((n_edges, width), wdt),
            jax.ShapeDtypeStruct((n_edges, width), wdt),
        ),
        mesh=_mesh(),
        scratch_types=scratch,
    )
    def gather2(ta, tb, ia, ib, oa, ob, idxa, idxb,
                a0, b0, a1, b1, sa0, sb0, sa1, sb1, *tails):
        wid = lax.axis_index("s") * NC + lax.axis_index("c")
        base0 = wid * per
        pltpu.sync_copy(ia.at[pl.ds(base0, per)], idxa)
        pltpu.sync_copy(ib.at[pl.ds(base0, per)], idxb)
        pairs = ((a0, b0, sa0, sb0), (a1, b1, sa1, sb1))

        def issue(k, p):
            ba, bb, sa, sb = p
            pltpu.async_copy(ta.at[idxa.at[pl.ds(k * CH, CH)]], ba, sa)
            pltpu.async_copy(tb.at[idxb.at[pl.ds(k * CH, CH)]], bb, sb)

        def wait(k, p):
            ba, bb, sa, sb = p
            pltpu.make_async_copy(
                ta.at[idxa.at[pl.ds(k * CH, CH)]], ba, sa).wait()
            pltpu.make_async_copy(
                tb.at[idxb.at[pl.ds(k * CH, CH)]], bb, sb).wait()

        def drain(k, p):
            ba, bb, sa, sb = p
            pltpu.sync_copy(ba, oa.at[pl.ds(base0 + k * CH, CH)])
            pltpu.sync_copy(bb, ob.at[pl.ds(base0 + k * CH, CH)])

        for b in range(2):
            issue(b, pairs[b])

        def group(g, c):
            for b in range(2):
                k = 2 * g + b
                p = pairs[b]
                wait(k, p)
                drain(k, p)
                issue(k + 2, p)
            return c

        lax.fori_loop(0, ngroups - 1, group, 0)
        for b in range(2):
            k = 2 * (ngroups - 1) + b
            p = pairs[b]
            wait(k, p)
            drain(k, p)
        if tail:
            tba, tbb = tails
            kb = n_full * CH
            pltpu.async_copy(
                ta.at[idxa.at[pl.ds(kb, tail)]], tba, sa0).wait()
            pltpu.async_copy(
                tb.at[idxb.at[pl.ds(kb, tail)]], tbb, sb0).wait()
            pltpu.sync_copy(tba, oa.at[pl.ds(base0 + kb, tail)])
            pltpu.sync_copy(tbb, ob.at[pl.ds(base0 + kb, tail)])

    return gather2


@functools.cache
def _make_segsum(n_edges, width):
    half = width // NC
    per = n_edges // NS
    assert per * NS == n_edges
    CH = 128
    n_full = per // CH
    tail = per - n_full * CH
    ngroups = n_full // 2
    assert n_full % 2 == 0 and tail % 8 == 0

    scratch = [
        pltpu.VMEM((CH,), jnp.int32),
        pltpu.VMEM((CH, half), F32),
        pltpu.VMEM((CH,), jnp.int32),
        pltpu.VMEM((CH, half), F32),
        pltpu.VMEM_SHARED((NPAD, half), F32),
        pltpu.SemaphoreType.DMA,
        pltpu.SemaphoreType.DMA,
        pltpu.SemaphoreType.DMA,
        pltpu.SemaphoreType.DMA,
    ]
    if tail:
        scratch += [
            pltpu.VMEM((tail,), jnp.int32),
            pltpu.VMEM((tail, half), F32),
        ]

    @functools.partial(
        pl.kernel,
        out_type=jax.ShapeDtypeStruct((NPAD, width), F32),
        mesh=_mesh(),
        scratch_types=scratch,
    )
    def segsum(vals, rows, zeros, out, i0, v0, i1, v1, acc,
               si0, sv0, si1, sv1, *tails):
        cid = lax.axis_index("c")
        sid = lax.axis_index("s")
        pltpu.sync_copy(
            zeros.at[pl.ds(sid * RPS, RPS), pl.ds(0, half)],
            acc.at[pl.ds(sid * RPS, RPS)],
        )
        plsc.subcore_barrier()

        base0 = sid * per
        pairs = ((i0, v0, si0, sv0), (i1, v1, si1, sv1))

        def issue(k, p):
            iv, vv, si, sv = p
            pltpu.async_copy(rows.at[pl.ds(base0 + k * CH, CH)], iv, si)
            pltpu.async_copy(
                vals.at[pl.ds(base0 + k * CH, CH), pl.ds(cid * half, half)],
                vv, sv)

        def wait(k, p):
            iv, vv, si, sv = p
            pltpu.make_async_copy(
                rows.at[pl.ds(base0 + k * CH, CH)], iv, si).wait()
            pltpu.make_async_copy(
                vals.at[pl.ds(base0 + k * CH, CH), pl.ds(cid * half, half)],
                vv, sv).wait()

        for b in range(2):
            issue(b, pairs[b])

        def group(g, c):
            for b in range(2):
                k = 2 * g + b
                p = pairs[b]
                wait(k, p)
                pltpu.sync_copy(p[1], acc.at[p[0]], add=True)
                issue(k + 2, p)
            return c

        lax.fori_loop(0, ngroups - 1, group, 0)
        for b in range(2):
            k = 2 * (ngroups - 1) + b
            p = pairs[b]
            wait(k, p)
            pltpu.sync_copy(p[1], acc.at[p[0]], add=True)
        if tail:
            ti, tv = tails
            kb = base0 + n_full * CH
            pltpu.sync_copy(rows.at[pl.ds(kb, tail)], ti)
            pltpu.sync_copy(
                vals.at[pl.ds(kb, tail), pl.ds(cid * half, half)], tv)
            pltpu.sync_copy(tv, acc.at[ti], add=True)

        plsc.subcore_barrier()
        pltpu.sync_copy(
            acc.at[pl.ds(sid * RPS, RPS)],
            out.at[pl.ds(sid * RPS, RPS), pl.ds(cid * half, half)],
        )

    return segsum


@functools.cache
def _make_segsum_part(n_edges):
    width = 128
    per_core = n_edges // NC
    per = per_core // NS
    CH = 64
    n_full = per // CH
    tail = per - n_full * CH
    ngroups = n_full // 2
    assert n_full % 2 == 0 and tail % 8 == 0

    scratch = [
        pltpu.VMEM((CH,), jnp.int32),
        pltpu.VMEM((CH, width), F32),
        pltpu.VMEM((CH,), jnp.int32),
        pltpu.VMEM((CH, width), F32),
        pltpu.VMEM_SHARED((NPAD, width), F32),
        pltpu.SemaphoreType.DMA,
        pltpu.SemaphoreType.DMA,
        pltpu.SemaphoreType.DMA,
        pltpu.SemaphoreType.DMA,
    ]
    if tail:
        scratch += [
            pltpu.VMEM((tail,), jnp.int32),
            pltpu.VMEM((tail, width), F32),
        ]

    @functools.partial(
        pl.kernel,
        out_type=jax.ShapeDtypeStruct((NC, NPAD, width), F32),
        mesh=_mesh(),
        scratch_types=scratch,
    )
    def segsum_part(vals, rows, zeros, out, i0, v0, i1, v1, acc,
                    si0, sv0, si1, sv1, *tails):
        cid = lax.axis_index("c")
        sid = lax.axis_index("s")
        pltpu.sync_copy(
            zeros.at[pl.ds(sid * RPS, RPS)],
            acc.at[pl.ds(sid * RPS, RPS)],
        )
        plsc.subcore_barrier()

        base0 = cid * per_core + sid * per
        pairs = ((i0, v0, si0, sv0), (i1, v1, si1, sv1))

        def issue(k, p):
            iv, vv, si, sv = p
            pltpu.async_copy(rows.at[pl.ds(base0 + k * CH, CH)], iv, si)
            pltpu.async_copy(vals.at[pl.ds(base0 + k * CH, CH)], vv, sv)

        def wait(k, p):
            iv, vv, si, sv = p
            pltpu.make_async_copy(
                rows.at[pl.ds(base0 + k * CH, CH)], iv, si).wait()
            pltpu.make_async_copy(
                vals.at[pl.ds(base0 + k * CH, CH)], vv, sv).wait()

        for b in range(2):
            issue(b, pairs[b])

        def group(g, c):
            for b in range(2):
                k = 2 * g + b
                p = pairs[b]
                wait(k, p)
                pltpu.sync_copy(p[1], acc.at[p[0]], add=True)
                issue(k + 2, p)
            return c

        lax.fori_loop(0, ngroups - 1, group, 0)
        for b in range(2):
            k = 2 * (ngroups - 1) + b
            p = pairs[b]
            wait(k, p)
            pltpu.sync_copy(p[1], acc.at[p[0]], add=True)
        if tail:
            ti, tv = tails
            kb = base0 + n_full * CH
            pltpu.sync_copy(rows.at[pl.ds(kb, tail)], ti)
            pltpu.sync_copy(vals.at[pl.ds(kb, tail)], tv)
            pltpu.sync_copy(tv, acc.at[ti], add=True)

        plsc.subcore_barrier()
        pltpu.sync_copy(
            acc.at[pl.ds(sid * RPS, RPS)],
            out.at[cid, pl.ds(sid * RPS, RPS)],
        )

    return segsum_part


def _silu(v):
    return v * jax.nn.sigmoid(v)


def _dot(a, b):
    return jnp.dot(a, b, preferred_element_type=F32)


BF16 = jnp.bfloat16


def _dotb(a, b):
    return jnp.dot(a.astype(BF16), b, preferred_element_type=F32)


_BN = 2000
_BE = 1600


def _full(shape):
    return pl.BlockSpec(shape, lambda i: (0,) * len(shape))


def _proj_body(h, wr, wc, hr, hc):
    hv = h[...]
    hr[...] = _dotb(hv, wr[...]).astype(BF16)
    hc[...] = _dotb(hv, wc[...]).astype(BF16)


def _proj(h, wr, wc):
    n, d = h.shape
    return pl.pallas_call(
        _proj_body,
        grid=(n // _BN,),
        in_specs=[
            pl.BlockSpec((_BN, d), lambda i: (i, 0)),
            _full((d, d)),
            _full((d, d)),
        ],
        out_specs=[pl.BlockSpec((_BN, d), lambda i: (i, 0))] * 2,
        out_shape=[jax.ShapeDtypeStruct((n, d), BF16)] * 2,
    )(h, wr, wc)


def _edge_body(gr, gc, diff, ea, wea, wrad, be1, we2, be2, out):
    d = diff[...][:, :16]
    radial = jnp.sum(d * d, axis=1, keepdims=True)
    pre = (
        (gr[...] + gc[...]).astype(F32) + _dot(ea[...], wea[...])
        + radial * wrad[...] + be1[...]
    )
    m = _silu(pre)
    out[...] = _silu(_dotb(m, we2[...]) + be2[...])


def _edge_last_body(gr, gc, diff, ea, wea, wrad, be1, we2, be2,
                    wc1, bc1, wc2t, out, trans):
    d = diff[...][:, :16]
    radial = jnp.sum(d * d, axis=1, keepdims=True)
    pre = (
        (gr[...] + gc[...]).astype(F32) + _dot(ea[...], wea[...])
        + radial * wrad[...] + be1[...]
    )
    m = _silu(pre)
    m2 = _silu(_dotb(m, we2[...]) + be2[...])
    out[...] = m2
    c1 = _silu(_dotb(m2, wc1[...]) + bc1[...])
    w = jnp.sum(c1 * wc2t[...], axis=1, keepdims=True)
    trans[...] = jnp.concatenate(
        [d * w, jnp.zeros((d.shape[0], 112), F32)], axis=1
    )


def _edge_mlp(gr, gc, diff128, ea, wea, wrad, be1, we2, be2, coord=None):
    e, d = gr.shape
    de = ea.shape[1]
    edge_spec = pl.BlockSpec((_BE, d), lambda i: (i, 0))
    diff_spec = pl.BlockSpec((_BE, 128), lambda i: (i, 0))
    ea_spec = pl.BlockSpec((_BE, de), lambda i: (i, 0))
    in_specs = [
        edge_spec, edge_spec, diff_spec, ea_spec,
        _full((de, d)), _full((1, d)), _full((1, d)),
        _full((d, d)), _full((1, d)),
    ]
    args = [gr, gc, diff128, ea, wea, wrad, be1, we2, be2]
    if coord is None:
        return pl.pallas_call(
            _edge_body,
            grid=(e // _BE,),
            in_specs=in_specs,
            out_specs=edge_spec,
            out_shape=jax.ShapeDtypeStruct((e, d), F32),
        )(*args)
    wc1, bc1, wc2t = coord
    return pl.pallas_call(
        _edge_last_body,
        grid=(e // _BE,),
        in_specs=in_specs + [_full((d, d)), _full((1, d)), _full((1, d))],
        out_specs=[edge_spec, pl.BlockSpec((_BE, 128), lambda i: (i, 0))],
        out_shape=[
            jax.ShapeDtypeStruct((e, d), F32),
            jax.ShapeDtypeStruct((e, 128), F32),
        ],
    )(*args, wc1, bc1, wc2t)


def _node_body(h, agg, wn1h, wn1a, bn1, wn2, bn2, wrn, wcn,
               out_h, out_hr, out_hc):
    hv = h[...]
    t = _silu(_dotb(hv, wn1h[...]) + _dotb(agg[...], wn1a[...]) + bn1[...])
    hn = hv + _dotb(t, wn2[...]) + bn2[...]
    out_h[...] = hn
    out_hr[...] = _dotb(hn, wrn[...]).astype(BF16)
    out_hc[...] = _dotb(hn, wcn[...]).astype(BF16)


def _node_mlp(h, agg, wn1h, wn1a, bn1, wn2, bn2, wrn, wcn):
    n, d = h.shape
    node_spec = pl.BlockSpec((_BN, d), lambda i: (i, 0))
    return pl.pallas_call(
        _node_body,
        grid=(n // _BN,),
        in_specs=[
            node_spec, node_spec,
            _full((d, d)), _full((d, d)), _full((1, d)),
            _full((d, d)), _full((1, d)),
            _full((d, d)), _full((d, d)),
        ],
        out_specs=[node_spec] * 3,
        out_shape=[jax.ShapeDtypeStruct((n, d), F32),
                   jax.ShapeDtypeStruct((n, d), BF16),
                   jax.ShapeDtypeStruct((n, d), BF16)],
    )(h, agg, wn1h, wn1a, bn1, wn2, bn2, wrn, wcn)


def _node_last_body(h, agg, x16, p0, p1, wn1h, wn1a, bn1, wn2, bn2,
                    out_h, out_x):
    hv = h[...]
    t = _silu(_dotb(hv, wn1h[...]) + _dotb(agg[...], wn1a[...]) + bn1[...])
    out_h[...] = hv + _dotb(t, wn2[...]) + bn2[...]
    out_x[...] = x16[...] + p0[...] + p1[...]


def _node_mlp_last(h, agg, x16, p0, p1, wn1h, wn1a, bn1, wn2, bn2):
    n, d = h.shape
    node_spec = pl.BlockSpec((_BN, d), lambda i: (i, 0))
    nar_spec = pl.BlockSpec((_BN, 16), lambda i: (i, 0))
    return pl.pallas_call(
        _node_last_body,
        grid=(n // _BN,),
        in_specs=[
            node_spec, node_spec, nar_spec, nar_spec, nar_spec,
            _full((d, d)), _full((d, d)), _full((1, d)),
            _full((d, d)), _full((1, d)),
        ],
        out_specs=[node_spec, nar_spec],
        out_shape=[
            jax.ShapeDtypeStruct((n, d), F32),
            jax.ShapeDtypeStruct((n, 16), F32),
        ],
    )(h, agg, x16, p0, p1, wn1h, wn1a, bn1, wn2, bn2)


def _pack_bf16(a):
    return lax.bitcast_convert_type(
        a.reshape(a.shape[0], -1, 2), jnp.int32)


def _unpack_bf16(a):
    return lax.bitcast_convert_type(a, jnp.bfloat16).reshape(a.shape[0], -1)


def kernel(h, x, edges, edge_attr, params):
    layers = params["layers"]
    n, d = h.shape
    e = edges.shape[1]
    de = edge_attr.shape[1]
    row = edges[0]
    col = edges[1]

    def split(p, with_coord):
        we1 = p["We1"]
        out = {
            "wr": we1[:d].astype(BF16),
            "wc": we1[d:2 * d].astype(BF16),
            "wrad": we1[2 * d:2 * d + 1],
            "wea": we1[2 * d + 1:],
            "be1": p["be1"].reshape(1, d),
            "we2": p["We2"].astype(BF16),
            "be2": p["be2"].reshape(1, d),
            "wn1h": p["Wn1"][:d].astype(BF16),
            "wn1a": p["Wn1"][d:].astype(BF16),
            "bn1": p["bn1"].reshape(1, d),
            "wn2": p["Wn2"].astype(BF16),
            "bn2": p["bn2"].reshape(1, d),
        }
        if with_coord:
            out["wc1"] = p["Wc1"].astype(BF16)
            out["bc1"] = p["bc1"].reshape(1, d)
            out["wc2t"] = p["Wc2"].reshape(1, d)
        return out

    nl = len(layers)
    ps = [split(p, i == nl - 1) for i, p in enumerate(layers)]

    x16 = jnp.pad(x, ((0, 0), (0, 16 - x.shape[1])))
    x128 = jnp.pad(x, ((0, 0), (0, 128 - x.shape[1])))
    zeros128 = jnp.zeros((NPAD, 128), F32)

    gdiff = _make_gather2sum(n, 128, e, True)
    g2 = _make_gather2(n, d // 2, e, jnp.int32)
    segsum = _make_segsum(e, d)
    segsum_part = _make_segsum_part(e)

    diff128 = gdiff(x128, x128, row, col)

    hr, hc = _proj(h, ps[0]["wr"], ps[0]["wc"])
    for i, p in enumerate(ps):
        gra, grb = g2(_pack_bf16(hr), _pack_bf16(hc), row, col)
        gr, gc = _unpack_bf16(gra), _unpack_bf16(grb)
        if i < nl - 1:
            m2 = _edge_mlp(gr, gc, diff128, edge_attr,
                           p["wea"], p["wrad"], p["be1"],
                           p["we2"], p["be2"])
            agg = segsum(m2, row, zeros128)
            h, hr, hc = _node_mlp(h, agg,
                                  p["wn1h"], p["wn1a"], p["bn1"],
                                  p["wn2"], p["bn2"],
                                  ps[i + 1]["wr"], ps[i + 1]["wc"])
        else:
            m2, trans = _edge_mlp(gr, gc, diff128, edge_attr,
                                  p["wea"], p["wrad"], p["be1"],
                                  p["we2"], p["be2"],
                                  coord=(p["wc1"], p["bc1"], p["wc2t"]))
            agg = segsum(m2, row, zeros128)
            parts = segsum_part(trans, row, zeros128)
            h, x16o = _node_mlp_last(h, agg, x16,
                                     parts[0, :n, :16], parts[1, :n, :16],
                                     p["wn1h"], p["wn1a"], p["bn1"],
                                     p["wn2"], p["bn2"])
    return (h, x16o[:, :3])


# speedup vs baseline: 3.8698x; 3.8698x over previous
import functools

import jax
import jax.numpy as jnp
from jax import lax
from jax.experimental import pallas as pl
from jax.experimental.pallas import tpu as pltpu
from jax.experimental.pallas import tpu_sc as plsc

F32 = jnp.float32

NC = 2
NS = 16
NW = NC * NS
NPAD = 10240
RPS = NPAD // NS


@functools.cache
def _mesh():
    return plsc.VectorSubcoreMesh(
        core_axis_name="c", subcore_axis_name="s",
        num_cores=NC, num_subcores=NS,
    )


@functools.cache
def _make_gather2sum(n_rows, width, n_edges, subtract, dtype=F32):
    per = n_edges // NW
    assert per * NW == n_edges
    CH = 64
    n_full = per // CH
    tail = per - n_full * CH
    ngroups = n_full // 2
    assert n_full % 2 == 0 and tail % 8 == 0
    isbf = dtype == jnp.bfloat16
    wdt = jnp.int32 if isbf else F32
    nslice = width // 16

    scratch = [
        pltpu.VMEM((per,), jnp.int32),
        pltpu.VMEM((per,), jnp.int32),
    ]
    for _ in range(2):
        scratch += [pltpu.VMEM((CH, width), wdt)] * 3
    scratch += [pltpu.SemaphoreType.DMA] * 4
    if tail:
        scratch += [pltpu.VMEM((tail, width), wdt)] * 3

    @functools.partial(
        pl.kernel,
        out_type=jax.ShapeDtypeStruct((n_edges, width), wdt),
        mesh=_mesh(),
        scratch_types=scratch,
    )
    def gather2sum(ta, tb, ia, ib, out, idxa, idxb,
                   a0, b0, o0, a1, b1, o1, sa0, sb0, sa1, sb1, *tails):
        wid = lax.axis_index("s") * NC + lax.axis_index("c")
        base0 = wid * per
        pltpu.sync_copy(ia.at[pl.ds(base0, per)], idxa)
        pltpu.sync_copy(ib.at[pl.ds(base0, per)], idxb)
        pairs = ((a0, b0, o0, sa0, sb0), (a1, b1, o1, sa1, sb1))

        def issue(k, p):
            ba, bb, _, sa, sb = p
            pltpu.async_copy(ta.at[idxa.at[pl.ds(k * CH, CH)]], ba, sa)
            pltpu.async_copy(tb.at[idxb.at[pl.ds(k * CH, CH)]], bb, sb)

        def wait(k, p):
            ba, bb, _, sa, sb = p
            pltpu.make_async_copy(
                ta.at[idxa.at[pl.ds(k * CH, CH)]], ba, sa).wait()
            pltpu.make_async_copy(
                tb.at[idxb.at[pl.ds(k * CH, CH)]], bb, sb).wait()

        def combine(ba, bb, bo, n):
            def rowbody(r, c):
                for j in range(nslice):
                    sl = (r, pl.ds(j * 16, 16))
                    if isbf:
                        va = plsc.bitcast(ba[sl], jnp.bfloat16)
                        vb = plsc.bitcast(bb[sl], jnp.bfloat16)
                        v = va - vb if subtract else va + vb
                        bo[sl] = plsc.bitcast(v, jnp.int32)
                    else:
                        v = ba[sl] - bb[sl] if subtract else ba[sl] + bb[sl]
                        bo[sl] = v
                return c
            lax.fori_loop(0, n, rowbody, 0)

        for b in range(2):
            issue(b, pairs[b])

        def group(g, c):
            for b in range(2):
                k = 2 * g + b
                p = pairs[b]
                wait(k, p)
                combine(p[0], p[1], p[2], CH)
                issue(k + 2, p)
                pltpu.sync_copy(p[2], out.at[pl.ds(base0 + k * CH, CH)])
            return c

        lax.fori_loop(0, ngroups - 1, group, 0)
        for b in range(2):
            k = 2 * (ngroups - 1) + b
            p = pairs[b]
            wait(k, p)
            combine(p[0], p[1], p[2], CH)
            pltpu.sync_copy(p[2], out.at[pl.ds(base0 + k * CH, CH)])
        if tail:
            tba, tbb, tbo = tails
            kb = n_full * CH
            pltpu.async_copy(
                ta.at[idxa.at[pl.ds(kb, tail)]], tba, sa0).wait()
            pltpu.async_copy(
                tb.at[idxb.at[pl.ds(kb, tail)]], tbb, sb0).wait()
            combine(tba, tbb, tbo, tail)
            pltpu.sync_copy(tbo, out.at[pl.ds(base0 + kb, tail)])

    return gather2sum


@functools.cache
def _make_gather2(n_rows, width, n_edges, wdt):
    per = n_edges // NW
    assert per * NW == n_edges
    CH = 64
    n_full = per // CH
    tail = per - n_full * CH
    ngroups = n_full // 2
    assert n_full % 2 == 0 and tail % 8 == 0

    scratch = [
        pltpu.VMEM((per,), jnp.int32),
        pltpu.VMEM((per,), jnp.int32),
    ]
    for _ in range(2):
        scratch += [pltpu.VMEM((CH, width), wdt)] * 2
    scratch += [pltpu.SemaphoreType.DMA] * 4
    if tail:
        scratch += [pltpu.VMEM((tail, width), wdt)] * 2

    @functools.partial(
        pl.kernel,
        out_type=(
            jax.ShapeDtypeStruct((n_edges, width), wdt),
            jax.ShapeDtypeStruct((n_edges, width), wdt),
        ),
        mesh=_mesh(),
        scratch_types=scratch,
    )
    def gather2(ta, tb, ia, ib, oa, ob, idxa, idxb,
                a0, b0, a1, b1, sa0, sb0, sa1, sb1, *tails):
        wid = lax.axis_index("s") * NC + lax.axis_index("c")
        base0 = wid * per
        pltpu.sync_copy(ia.at[pl.ds(base0, per)], idxa)
        pltpu.sync_copy(ib.at[pl.ds(base0, per)], idxb)
        pairs = ((a0, b0, sa0, sb0), (a1, b1, sa1, sb1))

        def issue(k, p):
            ba, bb, sa, sb = p
            pltpu.async_copy(ta.at[idxa.at[pl.ds(k * CH, CH)]], ba, sa)
            pltpu.async_copy(tb.at[idxb.at[pl.ds(k * CH, CH)]], bb, sb)

        def wait(k, p):
            ba, bb, sa, sb = p
            pltpu.make_async_copy(
                ta.at[idxa.at[pl.ds(k * CH, CH)]], ba, sa).wait()
            pltpu.make_async_copy(
                tb.at[idxb.at[pl.ds(k * CH, CH)]], bb, sb).wait()

        def drain(k, p):
            ba, bb, sa, sb = p
            pltpu.sync_copy(ba, oa.at[pl.ds(base0 + k * CH, CH)])
            pltpu.sync_copy(bb, ob.at[pl.ds(base0 + k * CH, CH)])

        for b in range(2):
            issue(b, pairs[b])

        def group(g, c):
            for b in range(2):
                k = 2 * g + b
                p = pairs[b]
                wait(k, p)
                drain(k, p)
                issue(k + 2, p)
            return c

        lax.fori_loop(0, ngroups - 1, group, 0)
        for b in range(2):
            k = 2 * (ngroups - 1) + b
            p = pairs[b]
            wait(k, p)
            drain(k, p)
        if tail:
            tba, tbb = tails
            kb = n_full * CH
            pltpu.async_copy(
                ta.at[idxa.at[pl.ds(kb, tail)]], tba, sa0).wait()
            pltpu.async_copy(
                tb.at[idxb.at[pl.ds(kb, tail)]], tbb, sb0).wait()
            pltpu.sync_copy(tba, oa.at[pl.ds(base0 + kb, tail)])
            pltpu.sync_copy(tbb, ob.at[pl.ds(base0 + kb, tail)])

    return gather2


@functools.cache
def _make_segsum(n_edges, width):
    half = width // NC
    per = n_edges // NS
    assert per * NS == n_edges
    CH = 128
    n_full = per // CH
    tail = per - n_full * CH
    ngroups = n_full // 2
    assert n_full % 2 == 0 and tail % 8 == 0

    scratch = [
        pltpu.VMEM((CH,), jnp.int32),
        pltpu.VMEM((CH, half), F32),
        pltpu.VMEM((CH,), jnp.int32),
        pltpu.VMEM((CH, half), F32),
        pltpu.VMEM_SHARED((NPAD, half), F32),
        pltpu.SemaphoreType.DMA,
        pltpu.SemaphoreType.DMA,
        pltpu.SemaphoreType.DMA,
        pltpu.SemaphoreType.DMA,
    ]
    if tail:
        scratch += [
            pltpu.VMEM((tail,), jnp.int32),
            pltpu.VMEM((tail, half), F32),
        ]

    @functools.partial(
        pl.kernel,
        out_type=jax.ShapeDtypeStruct((NPAD, width), F32),
        mesh=_mesh(),
        scratch_types=scratch,
    )
    def segsum(vals, rows, zeros, out, i0, v0, i1, v1, acc,
               si0, sv0, si1, sv1, *tails):
        cid = lax.axis_index("c")
        sid = lax.axis_index("s")
        pltpu.sync_copy(
            zeros.at[pl.ds(sid * RPS, RPS), pl.ds(0, half)],
            acc.at[pl.ds(sid * RPS, RPS)],
        )
        plsc.subcore_barrier()

        base0 = sid * per
        pairs = ((i0, v0, si0, sv0), (i1, v1, si1, sv1))

        def issue(k, p):
            iv, vv, si, sv = p
            pltpu.async_copy(rows.at[pl.ds(base0 + k * CH, CH)], iv, si)
            pltpu.async_copy(
                vals.at[pl.ds(base0 + k * CH, CH), pl.ds(cid * half, half)],
                vv, sv)

        def wait(k, p):
            iv, vv, si, sv = p
            pltpu.make_async_copy(
                rows.at[pl.ds(base0 + k * CH, CH)], iv, si).wait()
            pltpu.make_async_copy(
                vals.at[pl.ds(base0 + k * CH, CH), pl.ds(cid * half, half)],
                vv, sv).wait()

        for b in range(2):
            issue(b, pairs[b])

        def group(g, c):
            for b in range(2):
                k = 2 * g + b
                p = pairs[b]
                wait(k, p)
                pltpu.sync_copy(p[1], acc.at[p[0]], add=True)
                issue(k + 2, p)
            return c

        lax.fori_loop(0, ngroups - 1, group, 0)
        for b in range(2):
            k = 2 * (ngroups - 1) + b
            p = pairs[b]
            wait(k, p)
            pltpu.sync_copy(p[1], acc.at[p[0]], add=True)
        if tail:
            ti, tv = tails
            kb = base0 + n_full * CH
            pltpu.sync_copy(rows.at[pl.ds(kb, tail)], ti)
            pltpu.sync_copy(
                vals.at[pl.ds(kb, tail), pl.ds(cid * half, half)], tv)
            pltpu.sync_copy(tv, acc.at[ti], add=True)

        plsc.subcore_barrier()
        pltpu.sync_copy(
            acc.at[pl.ds(sid * RPS, RPS)],
            out.at[pl.ds(sid * RPS, RPS), pl.ds(cid * half, half)],
        )

    return segsum


@functools.cache
def _make_segsum_part(n_edges):
    width = 128
    per_core = n_edges // NC
    per = per_core // NS
    CH = 64
    n_full = per // CH
    tail = per - n_full * CH
    ngroups = n_full // 2
    assert n_full % 2 == 0 and tail % 8 == 0

    scratch = [
        pltpu.VMEM((CH,), jnp.int32),
        pltpu.VMEM((CH, width), F32),
        pltpu.VMEM((CH,), jnp.int32),
        pltpu.VMEM((CH, width), F32),
        pltpu.VMEM_SHARED((NPAD, width), F32),
        pltpu.SemaphoreType.DMA,
        pltpu.SemaphoreType.DMA,
        pltpu.SemaphoreType.DMA,
        pltpu.SemaphoreType.DMA,
    ]
    if tail:
        scratch += [
            pltpu.VMEM((tail,), jnp.int32),
            pltpu.VMEM((tail, width), F32),
        ]

    @functools.partial(
        pl.kernel,
        out_type=jax.ShapeDtypeStruct((NC, NPAD, width), F32),
        mesh=_mesh(),
        scratch_types=scratch,
    )
    def segsum_part(vals, rows, zeros, out, i0, v0, i1, v1, acc,
                    si0, sv0, si1, sv1, *tails):
        cid = lax.axis_index("c")
        sid = lax.axis_index("s")
        pltpu.sync_copy(
            zeros.at[pl.ds(sid * RPS, RPS)],
            acc.at[pl.ds(sid * RPS, RPS)],
        )
        plsc.subcore_barrier()

        base0 = cid * per_core + sid * per
        pairs = ((i0, v0, si0, sv0), (i1, v1, si1, sv1))

        def issue(k, p):
            iv, vv, si, sv = p
            pltpu.async_copy(rows.at[pl.ds(base0 + k * CH, CH)], iv, si)
            pltpu.async_copy(vals.at[pl.ds(base0 + k * CH, CH)], vv, sv)

        def wait(k, p):
            iv, vv, si, sv = p
            pltpu.make_async_copy(
                rows.at[pl.ds(base0 + k * CH, CH)], iv, si).wait()
            pltpu.make_async_copy(
                vals.at[pl.ds(base0 + k * CH, CH)], vv, sv).wait()

        for b in range(2):
            issue(b, pairs[b])

        def group(g, c):
            for b in range(2):
                k = 2 * g + b
                p = pairs[b]
                wait(k, p)
                pltpu.sync_copy(p[1], acc.at[p[0]], add=True)
                issue(k + 2, p)
            return c

        lax.fori_loop(0, ngroups - 1, group, 0)
        for b in range(2):
            k = 2 * (ngroups - 1) + b
            p = pairs[b]
            wait(k, p)
            pltpu.sync_copy(p[1], acc.at[p[0]], add=True)
        if tail:
            ti, tv = tails
            kb = base0 + n_full * CH
            pltpu.sync_copy(rows.at[pl.ds(kb, tail)], ti)
            pltpu.sync_copy(vals.at[pl.ds(kb, tail)], tv)
            pltpu.sync_copy(tv, acc.at[ti], add=True)

        plsc.subcore_barrier()
        pltpu.sync_copy(
            acc.at[pl.ds(sid * RPS, RPS)],
            out.at[cid, pl.ds(sid * RPS, RPS)],
        )

    return segsum_part


def _silu(v):
    return v * jax.nn.sigmoid(v)


def _dot(a, b):
    return jnp.dot(a, b, preferred_element_type=F32)


BF16 = jnp.bfloat16


def _dotb(a, b):
    return jnp.dot(a.astype(BF16), b, preferred_element_type=F32)


def _bf16_word(lo_f32, hi_f32):
    ulo = lax.bitcast_convert_type(lo_f32, jnp.uint32)
    uhi = lax.bitcast_convert_type(hi_f32, jnp.uint32)
    rlo = (ulo + 0x7FFF + ((ulo >> 16) & 1)) >> 16
    rhi = (uhi + 0x7FFF + ((uhi >> 16) & 1)) >> 16
    return lax.bitcast_convert_type(rlo | (rhi << 16), jnp.int32)


def _unpack_words(w32):
    u = lax.bitcast_convert_type(w32, jnp.uint32)
    lo = lax.bitcast_convert_type(u << 16, F32)
    hi = lax.bitcast_convert_type(u & jnp.uint32(0xFFFF0000), F32)
    return lo, hi


_BN = 2000
_BE = 1600


def _full(shape):
    return pl.BlockSpec(shape, lambda i: (0,) * len(shape))


def _proj_body(h, wrlo, wrhi, wclo, wchi, hr, hc):
    hv = h[...]
    hr[...] = _bf16_word(_dotb(hv, wrlo[...]), _dotb(hv, wrhi[...]))
    hc[...] = _bf16_word(_dotb(hv, wclo[...]), _dotb(hv, wchi[...]))


def _proj(h, wrlo, wrhi, wclo, wchi):
    n, d = h.shape
    hw = d // 2
    return pl.pallas_call(
        _proj_body,
        grid=(n // _BN,),
        in_specs=[
            pl.BlockSpec((_BN, d), lambda i: (i, 0)),
            _full((d, hw)), _full((d, hw)),
            _full((d, hw)), _full((d, hw)),
        ],
        out_specs=[pl.BlockSpec((_BN, hw), lambda i: (i, 0))] * 2,
        out_shape=[jax.ShapeDtypeStruct((n, hw), jnp.int32)] * 2,
    )(h, wrlo, wrhi, wclo, wchi)


def _edge_body(gr, gc, diff, ea, wea, wrad, be1, we2, be2, out):
    d = diff[...][:, :16]
    radial = jnp.sum(d * d, axis=1, keepdims=True)
    glo, ghi = _unpack_words(gr[...])
    clo, chi = _unpack_words(gc[...])
    msum = jnp.concatenate([glo + clo, ghi + chi], axis=1)
    pre = (
        msum + _dot(ea[...], wea[...])
        + radial * wrad[...] + be1[...]
    )
    m = _silu(pre)
    out[...] = _silu(_dotb(m, we2[...]) + be2[...])


def _edge_last_body(gr, gc, diff, ea, wea, wrad, be1, we2, be2,
                    wc1, bc1, wc2t, out, trans):
    d = diff[...][:, :16]
    radial = jnp.sum(d * d, axis=1, keepdims=True)
    glo, ghi = _unpack_words(gr[...])
    clo, chi = _unpack_words(gc[...])
    msum = jnp.concatenate([glo + clo, ghi + chi], axis=1)
    pre = (
        msum + _dot(ea[...], wea[...])
        + radial * wrad[...] + be1[...]
    )
    m = _silu(pre)
    m2 = _silu(_dotb(m, we2[...]) + be2[...])
    out[...] = m2
    c1 = _silu(_dotb(m2, wc1[...]) + bc1[...])
    w = jnp.sum(c1 * wc2t[...], axis=1, keepdims=True)
    trans[...] = jnp.concatenate(
        [d * w, jnp.zeros((d.shape[0], 112), F32)], axis=1
    )


def _edge_mlp(gr, gc, diff128, ea, wea, wrad, be1, we2, be2, coord=None):
    e = gr.shape[0]
    d = gr.shape[1] * 2
    de = ea.shape[1]
    edge_spec = pl.BlockSpec((_BE, d), lambda i: (i, 0))
    word_spec = pl.BlockSpec((_BE, d // 2), lambda i: (i, 0))
    diff_spec = pl.BlockSpec((_BE, 128), lambda i: (i, 0))
    ea_spec = pl.BlockSpec((_BE, de), lambda i: (i, 0))
    in_specs = [
        word_spec, word_spec, diff_spec, ea_spec,
        _full((de, d)), _full((1, d)), _full((1, d)),
        _full((d, d)), _full((1, d)),
    ]
    args = [gr, gc, diff128, ea, wea, wrad, be1, we2, be2]
    if coord is None:
        return pl.pallas_call(
            _edge_body,
            grid=(e // _BE,),
            in_specs=in_specs,
            out_specs=edge_spec,
            out_shape=jax.ShapeDtypeStruct((e, d), F32),
        )(*args)
    wc1, bc1, wc2t = coord
    return pl.pallas_call(
        _edge_last_body,
        grid=(e // _BE,),
        in_specs=in_specs + [_full((d, d)), _full((1, d)), _full((1, d))],
        out_specs=[edge_spec, pl.BlockSpec((_BE, 128), lambda i: (i, 0))],
        out_shape=[
            jax.ShapeDtypeStruct((e, d), F32),
            jax.ShapeDtypeStruct((e, 128), F32),
        ],
    )(*args, wc1, bc1, wc2t)


def _node_body(h, agg, wn1h, wn1a, bn1, wn2, bn2,
               wrlo, wrhi, wclo, wchi, out_h, out_hr, out_hc):
    hv = h[...]
    t = _silu(_dotb(hv, wn1h[...]) + _dotb(agg[...], wn1a[...]) + bn1[...])
    hn = hv + _dotb(t, wn2[...]) + bn2[...]
    out_h[...] = hn
    out_hr[...] = _bf16_word(_dotb(hn, wrlo[...]), _dotb(hn, wrhi[...]))
    out_hc[...] = _bf16_word(_dotb(hn, wclo[...]), _dotb(hn, wchi[...]))


def _node_mlp(h, agg, wn1h, wn1a, bn1, wn2, bn2, wrlo, wrhi, wclo, wchi):
    n, d = h.shape
    hw = d // 2
    node_spec = pl.BlockSpec((_BN, d), lambda i: (i, 0))
    word_spec = pl.BlockSpec((_BN, hw), lambda i: (i, 0))
    return pl.pallas_call(
        _node_body,
        grid=(n // _BN,),
        in_specs=[
            node_spec, node_spec,
            _full((d, d)), _full((d, d)), _full((1, d)),
            _full((d, d)), _full((1, d)),
            _full((d, hw)), _full((d, hw)),
            _full((d, hw)), _full((d, hw)),
        ],
        out_specs=[node_spec, word_spec, word_spec],
        out_shape=[jax.ShapeDtypeStruct((n, d), F32),
                   jax.ShapeDtypeStruct((n, hw), jnp.int32),
                   jax.ShapeDtypeStruct((n, hw), jnp.int32)],
    )(h, agg, wn1h, wn1a, bn1, wn2, bn2, wrlo, wrhi, wclo, wchi)


def _node_last_body(h, agg, x16, p0, p1, wn1h, wn1a, bn1, wn2, bn2,
                    out_h, out_x):
    hv = h[...]
    t = _silu(_dotb(hv, wn1h[...]) + _dotb(agg[...], wn1a[...]) + bn1[...])
    out_h[...] = hv + _dotb(t, wn2[...]) + bn2[...]
    out_x[...] = x16[...] + p0[...] + p1[...]


def _node_mlp_last(h, agg, x16, p0, p1, wn1h, wn1a, bn1, wn2, bn2):
    n, d = h.shape
    node_spec = pl.BlockSpec((_BN, d), lambda i: (i, 0))
    nar_spec = pl.BlockSpec((_BN, 16), lambda i: (i, 0))
    return pl.pallas_call(
        _node_last_body,
        grid=(n // _BN,),
        in_specs=[
            node_spec, node_spec, nar_spec, nar_spec, nar_spec,
            _full((d, d)), _full((d, d)), _full((1, d)),
            _full((d, d)), _full((1, d)),
        ],
        out_specs=[node_spec, nar_spec],
        out_shape=[
            jax.ShapeDtypeStruct((n, d), F32),
            jax.ShapeDtypeStruct((n, 16), F32),
        ],
    )(h, agg, x16, p0, p1, wn1h, wn1a, bn1, wn2, bn2)


def kernel(h, x, edges, edge_attr, params):
    layers = params["layers"]
    n, d = h.shape
    e = edges.shape[1]
    de = edge_attr.shape[1]
    row = edges[0]
    col = edges[1]

    def split(p, with_coord):
        we1 = p["We1"]
        out = {
            "wrlo": we1[:d, :d // 2].astype(BF16),
            "wrhi": we1[:d, d // 2:].astype(BF16),
            "wclo": we1[d:2 * d, :d // 2].astype(BF16),
            "wchi": we1[d:2 * d, d // 2:].astype(BF16),
            "wrad": we1[2 * d:2 * d + 1],
            "wea": we1[2 * d + 1:],
            "be1": p["be1"].reshape(1, d),
            "we2": p["We2"].astype(BF16),
            "be2": p["be2"].reshape(1, d),
            "wn1h": p["Wn1"][:d].astype(BF16),
            "wn1a": p["Wn1"][d:].astype(BF16),
            "bn1": p["bn1"].reshape(1, d),
            "wn2": p["Wn2"].astype(BF16),
            "bn2": p["bn2"].reshape(1, d),
        }
        if with_coord:
            out["wc1"] = p["Wc1"].astype(BF16)
            out["bc1"] = p["bc1"].reshape(1, d)
            out["wc2t"] = p["Wc2"].reshape(1, d)
        return out

    nl = len(layers)
    ps = [split(p, i == nl - 1) for i, p in enumerate(layers)]

    x16 = jnp.pad(x, ((0, 0), (0, 16 - x.shape[1])))
    x128 = jnp.pad(x, ((0, 0), (0, 128 - x.shape[1])))
    zeros128 = jnp.zeros((NPAD, 128), F32)

    gdiff = _make_gather2sum(n, 128, e, True)
    g2 = _make_gather2(n, d // 2, e, jnp.int32)
    segsum = _make_segsum(e, d)
    segsum_part = _make_segsum_part(e)

    diff128 = gdiff(x128, x128, row, col)

    hr, hc = _proj(h, ps[0]["wrlo"], ps[0]["wrhi"],
                   ps[0]["wclo"], ps[0]["wchi"])
    for i, p in enumerate(ps):
        gr, gc = g2(hr, hc, row, col)
        if i < nl - 1:
            m2 = _edge_mlp(gr, gc, diff128, edge_attr,
                           p["wea"], p["wrad"], p["be1"],
                           p["we2"], p["be2"])
            agg = segsum(m2, row, zeros128)
            pn = ps[i + 1]
            h, hr, hc = _node_mlp(h, agg,
                                  p["wn1h"], p["wn1a"], p["bn1"],
                                  p["wn2"], p["bn2"],
                                  pn["wrlo"], pn["wrhi"],
                                  pn["wclo"], pn["wchi"])
        else:
            m2, trans = _edge_mlp(gr, gc, diff128, edge_attr,
                                  p["wea"], p["wrad"], p["be1"],
                                  p["we2"], p["be2"],
                                  coord=(p["wc1"], p["bc1"], p["wc2t"]))
            agg = segsum(m2, row, zeros128)
            parts = segsum_part(trans, row, zeros128)
            h, x16o = _node_mlp_last(h, agg, x16,
                                     parts[0, :n, :16], parts[1, :n, :16],
                                     p["wn1h"], p["wn1a"], p["bn1"],
                                     p["wn2"], p["bn2"])
    return (h, x16o[:, :3])


# speedup vs baseline: 3.9468x; 1.0199x over previous
import functools

import jax
import jax.numpy as jnp
from jax import lax
from jax.experimental import pallas as pl
from jax.experimental.pallas import tpu as pltpu
from jax.experimental.pallas import tpu_sc as plsc

F32 = jnp.float32

NC = 2
NS = 16
NW = NC * NS
NPAD = 10240
RPS = NPAD // NS


@functools.cache
def _mesh():
    return plsc.VectorSubcoreMesh(
        core_axis_name="c", subcore_axis_name="s",
        num_cores=NC, num_subcores=NS,
    )


@functools.cache
def _make_gather2sum(n_rows, width, n_edges, subtract, dtype=F32):
    per = n_edges // NW
    assert per * NW == n_edges
    CH = 64
    n_full = per // CH
    tail = per - n_full * CH
    ngroups = n_full // 2
    assert n_full % 2 == 0 and tail % 8 == 0
    isbf = dtype == jnp.bfloat16
    wdt = jnp.int32 if isbf else F32
    nslice = width // 16

    scratch = [
        pltpu.VMEM((per,), jnp.int32),
        pltpu.VMEM((per,), jnp.int32),
    ]
    for _ in range(2):
        scratch += [pltpu.VMEM((CH, width), wdt)] * 3
    scratch += [pltpu.SemaphoreType.DMA] * 4
    if tail:
        scratch += [pltpu.VMEM((tail, width), wdt)] * 3

    @functools.partial(
        pl.kernel,
        out_type=jax.ShapeDtypeStruct((n_edges, width), wdt),
        mesh=_mesh(),
        scratch_types=scratch,
    )
    def gather2sum(ta, tb, ia, ib, out, idxa, idxb,
                   a0, b0, o0, a1, b1, o1, sa0, sb0, sa1, sb1, *tails):
        wid = lax.axis_index("s") * NC + lax.axis_index("c")
        base0 = wid * per
        pltpu.sync_copy(ia.at[pl.ds(base0, per)], idxa)
        pltpu.sync_copy(ib.at[pl.ds(base0, per)], idxb)
        pairs = ((a0, b0, o0, sa0, sb0), (a1, b1, o1, sa1, sb1))

        def issue(k, p):
            ba, bb, _, sa, sb = p
            pltpu.async_copy(ta.at[idxa.at[pl.ds(k * CH, CH)]], ba, sa)
            pltpu.async_copy(tb.at[idxb.at[pl.ds(k * CH, CH)]], bb, sb)

        def wait(k, p):
            ba, bb, _, sa, sb = p
            pltpu.make_async_copy(
                ta.at[idxa.at[pl.ds(k * CH, CH)]], ba, sa).wait()
            pltpu.make_async_copy(
                tb.at[idxb.at[pl.ds(k * CH, CH)]], bb, sb).wait()

        def combine(ba, bb, bo, n):
            def rowbody(r, c):
                for j in range(nslice):
                    sl = (r, pl.ds(j * 16, 16))
                    if isbf:
                        va = plsc.bitcast(ba[sl], jnp.bfloat16)
                        vb = plsc.bitcast(bb[sl], jnp.bfloat16)
                        v = va - vb if subtract else va + vb
                        bo[sl] = plsc.bitcast(v, jnp.int32)
                    else:
                        v = ba[sl] - bb[sl] if subtract else ba[sl] + bb[sl]
                        bo[sl] = v
                return c
            lax.fori_loop(0, n, rowbody, 0)

        for b in range(2):
            issue(b, pairs[b])

        def group(g, c):
            for b in range(2):
                k = 2 * g + b
                p = pairs[b]
                wait(k, p)
                combine(p[0], p[1], p[2], CH)
                issue(k + 2, p)
                pltpu.sync_copy(p[2], out.at[pl.ds(base0 + k * CH, CH)])
            return c

        lax.fori_loop(0, ngroups - 1, group, 0)
        for b in range(2):
            k = 2 * (ngroups - 1) + b
            p = pairs[b]
            wait(k, p)
            combine(p[0], p[1], p[2], CH)
            pltpu.sync_copy(p[2], out.at[pl.ds(base0 + k * CH, CH)])
        if tail:
            tba, tbb, tbo = tails
            kb = n_full * CH
            pltpu.async_copy(
                ta.at[idxa.at[pl.ds(kb, tail)]], tba, sa0).wait()
            pltpu.async_copy(
                tb.at[idxb.at[pl.ds(kb, tail)]], tbb, sb0).wait()
            combine(tba, tbb, tbo, tail)
            pltpu.sync_copy(tbo, out.at[pl.ds(base0 + kb, tail)])

    return gather2sum


@functools.cache
def _make_gather2(n_rows, width, n_edges, wdt):
    per = n_edges // NW
    assert per * NW == n_edges
    CH = 64
    n_full = per // CH
    tail = per - n_full * CH
    ngroups = n_full // 3
    assert n_full % 3 == 0 and tail % 8 == 0
    scratch = [
        pltpu.VMEM((per,), jnp.int32),
        pltpu.VMEM((per,), jnp.int32),
    ]
    for _ in range(3):
        scratch += [pltpu.VMEM((CH, width), wdt)] * 2
    scratch += [pltpu.SemaphoreType.DMA] * 6
    if tail:
        scratch += [pltpu.VMEM((tail, width), wdt)] * 2

    @functools.partial(
        pl.kernel,
        out_type=(
            jax.ShapeDtypeStruct((n_edges, width), wdt),
            jax.ShapeDtypeStruct((n_edges, width), wdt),
        ),
        mesh=_mesh(),
        scratch_types=scratch,
    )
    def gather2(ta, tb, ia, ib, oa, ob, idxa, idxb,
                a0, b0, a1, b1, a2, b2,
                sa0, sb0, sa1, sb1, sa2, sb2, *tails):
        wid = lax.axis_index("s") * NC + lax.axis_index("c")
        base0 = wid * per
        pltpu.sync_copy(ia.at[pl.ds(base0, per)], idxa)
        pltpu.sync_copy(ib.at[pl.ds(base0, per)], idxb)
        pairs = ((a0, b0, sa0, sb0), (a1, b1, sa1, sb1), (a2, b2, sa2, sb2))

        def issue(k, p):
            ba, bb, sa, sb = p
            pltpu.async_copy(ta.at[idxa.at[pl.ds(k * CH, CH)]], ba, sa)
            pltpu.async_copy(tb.at[idxb.at[pl.ds(k * CH, CH)]], bb, sb)

        def wait(k, p):
            ba, bb, sa, sb = p
            pltpu.make_async_copy(
                ta.at[idxa.at[pl.ds(k * CH, CH)]], ba, sa).wait()
            pltpu.make_async_copy(
                tb.at[idxb.at[pl.ds(k * CH, CH)]], bb, sb).wait()

        def drain(k, p):
            ba, bb, sa, sb = p
            pltpu.sync_copy(ba, oa.at[pl.ds(base0 + k * CH, CH)])
            pltpu.sync_copy(bb, ob.at[pl.ds(base0 + k * CH, CH)])

        for b in range(3):
            issue(b, pairs[b])

        def group(g, c):
            for b in range(3):
                k = 3 * g + b
                p = pairs[b]
                wait(k, p)
                drain(k, p)
                issue(k + 3, p)
            return c

        lax.fori_loop(0, ngroups - 1, group, 0)
        for b in range(3):
            k = 3 * (ngroups - 1) + b
            p = pairs[b]
            wait(k, p)
            drain(k, p)
        if tail:
            tba, tbb = tails
            kb = n_full * CH
            pltpu.async_copy(
                ta.at[idxa.at[pl.ds(kb, tail)]], tba, sa0).wait()
            pltpu.async_copy(
                tb.at[idxb.at[pl.ds(kb, tail)]], tbb, sb0).wait()
            pltpu.sync_copy(tba, oa.at[pl.ds(base0 + kb, tail)])
            pltpu.sync_copy(tbb, ob.at[pl.ds(base0 + kb, tail)])

    return gather2


@functools.cache
def _make_segsum(n_edges, width):
    half = width // NC
    per = n_edges // NS
    assert per * NS == n_edges
    CH = 104
    n_full = per // CH
    tail = per - n_full * CH
    ngroups = n_full // 3
    assert n_full % 3 == 0 and tail % 8 == 0

    scratch = [
        pltpu.VMEM((CH,), jnp.int32),
        pltpu.VMEM((CH, half), F32),
        pltpu.VMEM((CH,), jnp.int32),
        pltpu.VMEM((CH, half), F32),
        pltpu.VMEM((CH,), jnp.int32),
        pltpu.VMEM((CH, half), F32),
        pltpu.VMEM_SHARED((NPAD, half), F32),
    ] + [pltpu.SemaphoreType.DMA] * 6
    if tail:
        scratch += [
            pltpu.VMEM((tail,), jnp.int32),
            pltpu.VMEM((tail, half), F32),
        ]

    @functools.partial(
        pl.kernel,
        out_type=jax.ShapeDtypeStruct((NPAD, width), F32),
        mesh=_mesh(),
        scratch_types=scratch,
    )
    def segsum(vals, rows, zeros, out, i0, v0, i1, v1, i2, v2, acc,
               si0, sv0, si1, sv1, si2, sv2, *tails):
        cid = lax.axis_index("c")
        sid = lax.axis_index("s")
        pltpu.sync_copy(
            zeros.at[pl.ds(sid * RPS, RPS), pl.ds(0, half)],
            acc.at[pl.ds(sid * RPS, RPS)],
        )
        plsc.subcore_barrier()

        base0 = sid * per
        pairs = ((i0, v0, si0, sv0), (i1, v1, si1, sv1), (i2, v2, si2, sv2))

        def issue(k, p):
            iv, vv, si, sv = p
            pltpu.async_copy(rows.at[pl.ds(base0 + k * CH, CH)], iv, si)
            pltpu.async_copy(
                vals.at[pl.ds(base0 + k * CH, CH), pl.ds(cid * half, half)],
                vv, sv)

        def wait(k, p):
            iv, vv, si, sv = p
            pltpu.make_async_copy(
                rows.at[pl.ds(base0 + k * CH, CH)], iv, si).wait()
            pltpu.make_async_copy(
                vals.at[pl.ds(base0 + k * CH, CH), pl.ds(cid * half, half)],
                vv, sv).wait()

        for b in range(3):
            issue(b, pairs[b])

        def group(g, c):
            for b in range(3):
                k = 3 * g + b
                p = pairs[b]
                wait(k, p)
                pltpu.sync_copy(p[1], acc.at[p[0]], add=True)
                issue(k + 3, p)
            return c

        lax.fori_loop(0, ngroups - 1, group, 0)
        for b in range(3):
            k = 3 * (ngroups - 1) + b
            p = pairs[b]
            wait(k, p)
            pltpu.sync_copy(p[1], acc.at[p[0]], add=True)
        if tail:
            ti, tv = tails
            kb = base0 + n_full * CH
            pltpu.sync_copy(rows.at[pl.ds(kb, tail)], ti)
            pltpu.sync_copy(
                vals.at[pl.ds(kb, tail), pl.ds(cid * half, half)], tv)
            pltpu.sync_copy(tv, acc.at[ti], add=True)

        plsc.subcore_barrier()
        pltpu.sync_copy(
            acc.at[pl.ds(sid * RPS, RPS)],
            out.at[pl.ds(sid * RPS, RPS), pl.ds(cid * half, half)],
        )

    return segsum


@functools.cache
def _make_segsum_part(n_edges):
    width = 128
    per_core = n_edges // NC
    per = per_core // NS
    CH = 64
    n_full = per // CH
    tail = per - n_full * CH
    ngroups = n_full // 3
    assert n_full % 3 == 0 and tail % 8 == 0

    scratch = [
        pltpu.VMEM((CH,), jnp.int32),
        pltpu.VMEM((CH, width), F32),
        pltpu.VMEM((CH,), jnp.int32),
        pltpu.VMEM((CH, width), F32),
        pltpu.VMEM((CH,), jnp.int32),
        pltpu.VMEM((CH, width), F32),
        pltpu.VMEM_SHARED((NPAD, width), F32),
    ] + [pltpu.SemaphoreType.DMA] * 6
    if tail:
        scratch += [
            pltpu.VMEM((tail,), jnp.int32),
            pltpu.VMEM((tail, width), F32),
        ]

    @functools.partial(
        pl.kernel,
        out_type=jax.ShapeDtypeStruct((NC, NPAD, width), F32),
        mesh=_mesh(),
        scratch_types=scratch,
    )
    def segsum_part(vals, rows, zeros, out, i0, v0, i1, v1, i2, v2, acc,
                    si0, sv0, si1, sv1, si2, sv2, *tails):
        cid = lax.axis_index("c")
        sid = lax.axis_index("s")
        pltpu.sync_copy(
            zeros.at[pl.ds(sid * RPS, RPS)],
            acc.at[pl.ds(sid * RPS, RPS)],
        )
        plsc.subcore_barrier()

        base0 = cid * per_core + sid * per
        pairs = ((i0, v0, si0, sv0), (i1, v1, si1, sv1), (i2, v2, si2, sv2))

        def issue(k, p):
            iv, vv, si, sv = p
            pltpu.async_copy(rows.at[pl.ds(base0 + k * CH, CH)], iv, si)
            pltpu.async_copy(vals.at[pl.ds(base0 + k * CH, CH)], vv, sv)

        def wait(k, p):
            iv, vv, si, sv = p
            pltpu.make_async_copy(
                rows.at[pl.ds(base0 + k * CH, CH)], iv, si).wait()
            pltpu.make_async_copy(
                vals.at[pl.ds(base0 + k * CH, CH)], vv, sv).wait()

        for b in range(3):
            issue(b, pairs[b])

        def group(g, c):
            for b in range(3):
                k = 3 * g + b
                p = pairs[b]
                wait(k, p)
                pltpu.sync_copy(p[1], acc.at[p[0]], add=True)
                issue(k + 3, p)
            return c

        lax.fori_loop(0, ngroups - 1, group, 0)
        for b in range(3):
            k = 3 * (ngroups - 1) + b
            p = pairs[b]
            wait(k, p)
            pltpu.sync_copy(p[1], acc.at[p[0]], add=True)
        if tail:
            ti, tv = tails
            kb = base0 + n_full * CH
            pltpu.sync_copy(rows.at[pl.ds(kb, tail)], ti)
            pltpu.sync_copy(vals.at[pl.ds(kb, tail)], tv)
            pltpu.sync_copy(tv, acc.at[ti], add=True)

        plsc.subcore_barrier()
        pltpu.sync_copy(
            acc.at[pl.ds(sid * RPS, RPS)],
            out.at[cid, pl.ds(sid * RPS, RPS)],
        )

    return segsum_part


def _silu(v):
    return v * jax.nn.sigmoid(v)


def _dot(a, b):
    return jnp.dot(a, b, preferred_element_type=F32)


BF16 = jnp.bfloat16


def _dotb(a, b):
    return jnp.dot(a.astype(BF16), b, preferred_element_type=F32)


def _bf16_word(lo_f32, hi_f32):
    ulo = lax.bitcast_convert_type(lo_f32, jnp.uint32)
    uhi = lax.bitcast_convert_type(hi_f32, jnp.uint32)
    rlo = (ulo + 0x7FFF + ((ulo >> 16) & 1)) >> 16
    rhi = (uhi + 0x7FFF + ((uhi >> 16) & 1)) >> 16
    return lax.bitcast_convert_type(rlo | (rhi << 16), jnp.int32)


def _unpack_words(w32):
    u = lax.bitcast_convert_type(w32, jnp.uint32)
    lo = lax.bitcast_convert_type(u << 16, F32)
    hi = lax.bitcast_convert_type(u & jnp.uint32(0xFFFF0000), F32)
    return lo, hi


_BN = 2000
_BE = 1600


def _full(shape):
    return pl.BlockSpec(shape, lambda i: (0,) * len(shape))


def _proj_body(h, wrlo, wrhi, wclo, wchi, hr, hc):
    hv = h[...]
    hr[...] = _bf16_word(_dotb(hv, wrlo[...]), _dotb(hv, wrhi[...]))
    hc[...] = _bf16_word(_dotb(hv, wclo[...]), _dotb(hv, wchi[...]))


def _proj(h, wrlo, wrhi, wclo, wchi):
    n, d = h.shape
    hw = d // 2
    return pl.pallas_call(
        _proj_body,
        grid=(n // _BN,),
        in_specs=[
            pl.BlockSpec((_BN, d), lambda i: (i, 0)),
            _full((d, hw)), _full((d, hw)),
            _full((d, hw)), _full((d, hw)),
        ],
        out_specs=[pl.BlockSpec((_BN, hw), lambda i: (i, 0))] * 2,
        out_shape=[jax.ShapeDtypeStruct((n, hw), jnp.int32)] * 2,
    )(h, wrlo, wrhi, wclo, wchi)


def _edge_first_body(gr, gc, diff, ea, wea, wrad, be1, we2, be2,
                     out, rad8):
    d = diff[...][:, :16]
    radial = jnp.sum(d * d, axis=1, keepdims=True)
    rad8[...] = jnp.broadcast_to(radial, (radial.shape[0], 8))
    glo, ghi = _unpack_words(gr[...])
    clo, chi = _unpack_words(gc[...])
    msum = jnp.concatenate([glo + clo, ghi + chi], axis=1)
    pre = (
        msum + _dot(ea[...], wea[...])
        + radial * wrad[...] + be1[...]
    )
    m = _silu(pre)
    out[...] = _silu(_dotb(m, we2[...]) + be2[...])


def _edge_mid_body(gr, gc, rad, ea, wea, wrad, be1, we2, be2, out):
    radial = rad[...][:, :1]
    glo, ghi = _unpack_words(gr[...])
    clo, chi = _unpack_words(gc[...])
    msum = jnp.concatenate([glo + clo, ghi + chi], axis=1)
    pre = (
        msum + _dot(ea[...], wea[...])
        + radial * wrad[...] + be1[...]
    )
    m = _silu(pre)
    out[...] = _silu(_dotb(m, we2[...]) + be2[...])


def _edge_last_body(gr, gc, diff, ea, wea, wrad, be1, we2, be2,
                    wc1, bc1, wc2t, out, trans):
    d = diff[...][:, :16]
    radial = jnp.sum(d * d, axis=1, keepdims=True)
    glo, ghi = _unpack_words(gr[...])
    clo, chi = _unpack_words(gc[...])
    msum = jnp.concatenate([glo + clo, ghi + chi], axis=1)
    pre = (
        msum + _dot(ea[...], wea[...])
        + radial * wrad[...] + be1[...]
    )
    m = _silu(pre)
    m2 = _silu(_dotb(m, we2[...]) + be2[...])
    out[...] = m2
    c1 = _silu(_dotb(m2, wc1[...]) + bc1[...])
    w = jnp.sum(c1 * wc2t[...], axis=1, keepdims=True)
    trans[...] = jnp.concatenate(
        [d * w, jnp.zeros((d.shape[0], 112), F32)], axis=1
    )


def _edge_mlp(gr, gc, aux, ea, wea, wrad, be1, we2, be2,
              mode="mid", coord=None):
    e = gr.shape[0]
    d = gr.shape[1] * 2
    de = ea.shape[1]
    edge_spec = pl.BlockSpec((_BE, d), lambda i: (i, 0))
    word_spec = pl.BlockSpec((_BE, d // 2), lambda i: (i, 0))
    diff_spec = pl.BlockSpec((_BE, 128), lambda i: (i, 0))
    rad_spec = pl.BlockSpec((_BE, 8), lambda i: (i, 0))
    ea_spec = pl.BlockSpec((_BE, de), lambda i: (i, 0))
    aux_spec = rad_spec if mode == "mid" else diff_spec
    in_specs = [
        word_spec, word_spec, aux_spec, ea_spec,
        _full((de, d)), _full((1, d)), _full((1, d)),
        _full((d, d)), _full((1, d)),
    ]
    args = [gr, gc, aux, ea, wea, wrad, be1, we2, be2]
    if mode == "first":
        return pl.pallas_call(
            _edge_first_body,
            grid=(e // _BE,),
            in_specs=in_specs,
            out_specs=[edge_spec, rad_spec],
            out_shape=[
                jax.ShapeDtypeStruct((e, d), F32),
                jax.ShapeDtypeStruct((e, 8), F32),
            ],
        )(*args)
    if mode == "mid":
        return pl.pallas_call(
            _edge_mid_body,
            grid=(e // _BE,),
            in_specs=in_specs,
            out_specs=edge_spec,
            out_shape=jax.ShapeDtypeStruct((e, d), F32),
        )(*args)
    wc1, bc1, wc2t = coord
    return pl.pallas_call(
        _edge_last_body,
        grid=(e // _BE,),
        in_specs=in_specs + [_full((d, d)), _full((1, d)), _full((1, d))],
        out_specs=[edge_spec, pl.BlockSpec((_BE, 128), lambda i: (i, 0))],
        out_shape=[
            jax.ShapeDtypeStruct((e, d), F32),
            jax.ShapeDtypeStruct((e, 128), F32),
        ],
    )(*args, wc1, bc1, wc2t)


def _node_body(h, agg, wn1h, wn1a, bn1, wn2, bn2,
               wrlo, wrhi, wclo, wchi, out_h, out_hr, out_hc):
    hv = h[...]
    t = _silu(_dotb(hv, wn1h[...]) + _dotb(agg[...], wn1a[...]) + bn1[...])
    hn = hv + _dotb(t, wn2[...]) + bn2[...]
    out_h[...] = hn
    out_hr[...] = _bf16_word(_dotb(hn, wrlo[...]), _dotb(hn, wrhi[...]))
    out_hc[...] = _bf16_word(_dotb(hn, wclo[...]), _dotb(hn, wchi[...]))


def _node_mlp(h, agg, wn1h, wn1a, bn1, wn2, bn2, wrlo, wrhi, wclo, wchi):
    n, d = h.shape
    hw = d // 2
    node_spec = pl.BlockSpec((_BN, d), lambda i: (i, 0))
    word_spec = pl.BlockSpec((_BN, hw), lambda i: (i, 0))
    return pl.pallas_call(
        _node_body,
        grid=(n // _BN,),
        in_specs=[
            node_spec, node_spec,
            _full((d, d)), _full((d, d)), _full((1, d)),
            _full((d, d)), _full((1, d)),
            _full((d, hw)), _full((d, hw)),
            _full((d, hw)), _full((d, hw)),
        ],
        out_specs=[node_spec, word_spec, word_spec],
        out_shape=[jax.ShapeDtypeStruct((n, d), F32),
                   jax.ShapeDtypeStruct((n, hw), jnp.int32),
                   jax.ShapeDtypeStruct((n, hw), jnp.int32)],
    )(h, agg, wn1h, wn1a, bn1, wn2, bn2, wrlo, wrhi, wclo, wchi)


def _node_last_body(h, agg, x16, p0, p1, wn1h, wn1a, bn1, wn2, bn2,
                    out_h, out_x):
    hv = h[...]
    t = _silu(_dotb(hv, wn1h[...]) + _dotb(agg[...], wn1a[...]) + bn1[...])
    out_h[...] = hv + _dotb(t, wn2[...]) + bn2[...]
    out_x[...] = x16[...] + p0[...] + p1[...]


def _node_mlp_last(h, agg, x16, p0, p1, wn1h, wn1a, bn1, wn2, bn2):
    n, d = h.shape
    node_spec = pl.BlockSpec((_BN, d), lambda i: (i, 0))
    nar_spec = pl.BlockSpec((_BN, 16), lambda i: (i, 0))
    return pl.pallas_call(
        _node_last_body,
        grid=(n // _BN,),
        in_specs=[
            node_spec, node_spec, nar_spec, nar_spec, nar_spec,
            _full((d, d)), _full((d, d)), _full((1, d)),
            _full((d, d)), _full((1, d)),
        ],
        out_specs=[node_spec, nar_spec],
        out_shape=[
            jax.ShapeDtypeStruct((n, d), F32),
            jax.ShapeDtypeStruct((n, 16), F32),
        ],
    )(h, agg, x16, p0, p1, wn1h, wn1a, bn1, wn2, bn2)


def kernel(h, x, edges, edge_attr, params):
    layers = params["layers"]
    n, d = h.shape
    e = edges.shape[1]
    de = edge_attr.shape[1]
    row = edges[0]
    col = edges[1]

    def split(p, with_coord):
        we1 = p["We1"]
        out = {
            "wrlo": we1[:d, :d // 2].astype(BF16),
            "wrhi": we1[:d, d // 2:].astype(BF16),
            "wclo": we1[d:2 * d, :d // 2].astype(BF16),
            "wchi": we1[d:2 * d, d // 2:].astype(BF16),
            "wrad": we1[2 * d:2 * d + 1],
            "wea": we1[2 * d + 1:],
            "be1": p["be1"].reshape(1, d),
            "we2": p["We2"].astype(BF16),
            "be2": p["be2"].reshape(1, d),
            "wn1h": p["Wn1"][:d].astype(BF16),
            "wn1a": p["Wn1"][d:].astype(BF16),
            "bn1": p["bn1"].reshape(1, d),
            "wn2": p["Wn2"].astype(BF16),
            "bn2": p["bn2"].reshape(1, d),
        }
        if with_coord:
            out["wc1"] = p["Wc1"].astype(BF16)
            out["bc1"] = p["bc1"].reshape(1, d)
            out["wc2t"] = p["Wc2"].reshape(1, d)
        return out

    nl = len(layers)
    ps = [split(p, i == nl - 1) for i, p in enumerate(layers)]

    x16 = jnp.pad(x, ((0, 0), (0, 16 - x.shape[1])))
    x128 = jnp.pad(x, ((0, 0), (0, 128 - x.shape[1])))
    zeros128 = jnp.zeros((NPAD, 128), F32)

    gdiff = _make_gather2sum(n, 128, e, True)
    g2 = _make_gather2(n, d // 2, e, jnp.int32)
    segsum = _make_segsum(e, d)
    segsum_part = _make_segsum_part(e)

    diff128 = gdiff(x128, x128, row, col)

    hr, hc = _proj(h, ps[0]["wrlo"], ps[0]["wrhi"],
                   ps[0]["wclo"], ps[0]["wchi"])
    for i, p in enumerate(ps):
        gr, gc = g2(hr, hc, row, col)
        if i == 0:
            m2, rad8 = _edge_mlp(gr, gc, diff128, edge_attr,
                                 p["wea"], p["wrad"], p["be1"],
                                 p["we2"], p["be2"], mode="first")
            agg = segsum(m2, row, zeros128)
            pn = ps[i + 1]
            h, hr, hc = _node_mlp(h, agg,
                                  p["wn1h"], p["wn1a"], p["bn1"],
                                  p["wn2"], p["bn2"],
                                  pn["wrlo"], pn["wrhi"],
                                  pn["wclo"], pn["wchi"])
        elif i < nl - 1:
            m2 = _edge_mlp(gr, gc, rad8, edge_attr,
                           p["wea"], p["wrad"], p["be1"],
                           p["we2"], p["be2"], mode="mid")
            agg = segsum(m2, row, zeros128)
            pn = ps[i + 1]
            h, hr, hc = _node_mlp(h, agg,
                                  p["wn1h"], p["wn1a"], p["bn1"],
                                  p["wn2"], p["bn2"],
                                  pn["wrlo"], pn["wrhi"],
                                  pn["wclo"], pn["wchi"])
        else:
            m2, trans = _edge_mlp(gr, gc, diff128, edge_attr,
                                  p["wea"], p["wrad"], p["be1"],
                                  p["we2"], p["be2"], mode="last",
                                  coord=(p["wc1"], p["bc1"], p["wc2t"]))
            agg = segsum(m2, row, zeros128)
            parts = segsum_part(trans, row, zeros128)
            h, x16o = _node_mlp_last(h, agg, x16,
                                     parts[0, :n, :16], parts[1, :n, :16],
                                     p["wn1h"], p["wn1a"], p["bn1"],
                                     p["wn2"], p["bn2"])
    return (h, x16o[:, :3])


# speedup vs baseline: 4.1358x; 1.0479x over previous
import functools

import jax
import jax.numpy as jnp
from jax import lax
from jax.experimental import pallas as pl
from jax.experimental.pallas import tpu as pltpu
from jax.experimental.pallas import tpu_sc as plsc

F32 = jnp.float32

NC = 2
NS = 16
NW = NC * NS
NPAD = 10240
RPS = NPAD // NS


@functools.cache
def _mesh():
    return plsc.VectorSubcoreMesh(
        core_axis_name="c", subcore_axis_name="s",
        num_cores=NC, num_subcores=NS,
    )


@functools.cache
def _make_gather2sum(n_rows, width, n_edges, subtract, dtype=F32):
    per = n_edges // NW
    assert per * NW == n_edges
    CH = 64
    n_full = per // CH
    tail = per - n_full * CH
    ngroups = n_full // 3
    assert n_full % 3 == 0 and tail % 8 == 0
    isbf = dtype == jnp.bfloat16
    wdt = jnp.int32 if isbf else F32
    nslice = width // 16

    scratch = [
        pltpu.VMEM((per,), jnp.int32),
        pltpu.VMEM((per,), jnp.int32),
    ]
    for _ in range(3):
        scratch += [pltpu.VMEM((CH, width), wdt)] * 3
    scratch += [pltpu.SemaphoreType.DMA] * 6
    if tail:
        scratch += [pltpu.VMEM((tail, width), wdt)] * 3

    @functools.partial(
        pl.kernel,
        out_type=jax.ShapeDtypeStruct((n_edges, width), wdt),
        mesh=_mesh(),
        scratch_types=scratch,
    )
    def gather2sum(ta, tb, ia, ib, out, idxa, idxb,
                   a0, b0, o0, a1, b1, o1, a2, b2, o2,
                   sa0, sb0, sa1, sb1, sa2, sb2, *tails):
        wid = lax.axis_index("s") * NC + lax.axis_index("c")
        base0 = wid * per
        pltpu.sync_copy(ia.at[pl.ds(base0, per)], idxa)
        pltpu.sync_copy(ib.at[pl.ds(base0, per)], idxb)
        pairs = ((a0, b0, o0, sa0, sb0), (a1, b1, o1, sa1, sb1),
                 (a2, b2, o2, sa2, sb2))

        def issue(k, p):
            ba, bb, _, sa, sb = p
            pltpu.async_copy(ta.at[idxa.at[pl.ds(k * CH, CH)]], ba, sa)
            pltpu.async_copy(tb.at[idxb.at[pl.ds(k * CH, CH)]], bb, sb)

        def wait(k, p):
            ba, bb, _, sa, sb = p
            pltpu.make_async_copy(
                ta.at[idxa.at[pl.ds(k * CH, CH)]], ba, sa).wait()
            pltpu.make_async_copy(
                tb.at[idxb.at[pl.ds(k * CH, CH)]], bb, sb).wait()

        def combine(ba, bb, bo, n):
            def rowbody(r, c):
                for j in range(nslice):
                    sl = (r, pl.ds(j * 16, 16))
                    if isbf:
                        va = plsc.bitcast(ba[sl], jnp.bfloat16)
                        vb = plsc.bitcast(bb[sl], jnp.bfloat16)
                        v = va - vb if subtract else va + vb
                        bo[sl] = plsc.bitcast(v, jnp.int32)
                    else:
                        v = ba[sl] - bb[sl] if subtract else ba[sl] + bb[sl]
                        bo[sl] = v
                return c
            lax.fori_loop(0, n, rowbody, 0)

        for b in range(3):
            issue(b, pairs[b])

        def group(g, c):
            for b in range(3):
                k = 3 * g + b
                p = pairs[b]
                wait(k, p)
                combine(p[0], p[1], p[2], CH)
                issue(k + 3, p)
                pltpu.sync_copy(p[2], out.at[pl.ds(base0 + k * CH, CH)])
            return c

        lax.fori_loop(0, ngroups - 1, group, 0)
        for b in range(3):
            k = 3 * (ngroups - 1) + b
            p = pairs[b]
            wait(k, p)
            combine(p[0], p[1], p[2], CH)
            pltpu.sync_copy(p[2], out.at[pl.ds(base0 + k * CH, CH)])
        if tail:
            tba, tbb, tbo = tails
            kb = n_full * CH
            pltpu.async_copy(
                ta.at[idxa.at[pl.ds(kb, tail)]], tba, sa0).wait()
            pltpu.async_copy(
                tb.at[idxb.at[pl.ds(kb, tail)]], tbb, sb0).wait()
            combine(tba, tbb, tbo, tail)
            pltpu.sync_copy(tbo, out.at[pl.ds(base0 + kb, tail)])

    return gather2sum


@functools.cache
def _make_gather2(n_rows, width, n_edges, wdt):
    per = n_edges // NW
    assert per * NW == n_edges
    CH = 64
    n_full = per // CH
    tail = per - n_full * CH
    ngroups = n_full // 3
    assert n_full % 3 == 0 and tail % 8 == 0
    scratch = [
        pltpu.VMEM((per,), jnp.int32),
        pltpu.VMEM((per,), jnp.int32),
    ]
    for _ in range(3):
        scratch += [pltpu.VMEM((CH, width), wdt)] * 2
    scratch += [pltpu.SemaphoreType.DMA] * 6
    if tail:
        scratch += [pltpu.VMEM((tail, width), wdt)] * 2

    @functools.partial(
        pl.kernel,
        out_type=(
            jax.ShapeDtypeStruct((n_edges, width), wdt),
            jax.ShapeDtypeStruct((n_edges, width), wdt),
        ),
        mesh=_mesh(),
        scratch_types=scratch,
    )
    def gather2(ta, tb, ia, ib, oa, ob, idxa, idxb,
                a0, b0, a1, b1, a2, b2,
                sa0, sb0, sa1, sb1, sa2, sb2, *tails):
        wid = lax.axis_index("s") * NC + lax.axis_index("c")
        base0 = wid * per
        pltpu.sync_copy(ia.at[pl.ds(base0, per)], idxa)
        pltpu.sync_copy(ib.at[pl.ds(base0, per)], idxb)
        pairs = ((a0, b0, sa0, sb0), (a1, b1, sa1, sb1), (a2, b2, sa2, sb2))

        def issue(k, p):
            ba, bb, sa, sb = p
            pltpu.async_copy(ta.at[idxa.at[pl.ds(k * CH, CH)]], ba, sa)
            pltpu.async_copy(tb.at[idxb.at[pl.ds(k * CH, CH)]], bb, sb)

        def wait(k, p):
            ba, bb, sa, sb = p
            pltpu.make_async_copy(
                ta.at[idxa.at[pl.ds(k * CH, CH)]], ba, sa).wait()
            pltpu.make_async_copy(
                tb.at[idxb.at[pl.ds(k * CH, CH)]], bb, sb).wait()

        def drain(k, p):
            ba, bb, sa, sb = p
            pltpu.sync_copy(ba, oa.at[pl.ds(base0 + k * CH, CH)])
            pltpu.sync_copy(bb, ob.at[pl.ds(base0 + k * CH, CH)])

        for b in range(3):
            issue(b, pairs[b])

        def group(g, c):
            for b in range(3):
                k = 3 * g + b
                p = pairs[b]
                wait(k, p)
                drain(k, p)
                issue(k + 3, p)
            return c

        lax.fori_loop(0, ngroups - 1, group, 0)
        for b in range(3):
            k = 3 * (ngroups - 1) + b
            p = pairs[b]
            wait(k, p)
            drain(k, p)
        if tail:
            tba, tbb = tails
            kb = n_full * CH
            pltpu.async_copy(
                ta.at[idxa.at[pl.ds(kb, tail)]], tba, sa0).wait()
            pltpu.async_copy(
                tb.at[idxb.at[pl.ds(kb, tail)]], tbb, sb0).wait()
            pltpu.sync_copy(tba, oa.at[pl.ds(base0 + kb, tail)])
            pltpu.sync_copy(tbb, ob.at[pl.ds(base0 + kb, tail)])

    return gather2


@functools.cache
def _make_segsum(n_edges, width):
    half = width // NC
    per = n_edges // NS
    assert per * NS == n_edges
    CH = 104
    n_full = per // CH
    tail = per - n_full * CH
    ngroups = n_full // 3
    assert n_full % 3 == 0 and tail % 8 == 0

    scratch = [
        pltpu.VMEM((CH,), jnp.int32),
        pltpu.VMEM((CH, half), F32),
        pltpu.VMEM((CH,), jnp.int32),
        pltpu.VMEM((CH, half), F32),
        pltpu.VMEM((CH,), jnp.int32),
        pltpu.VMEM((CH, half), F32),
        pltpu.VMEM_SHARED((NPAD, half), F32),
    ] + [pltpu.SemaphoreType.DMA] * 6
    if tail:
        scratch += [
            pltpu.VMEM((tail,), jnp.int32),
            pltpu.VMEM((tail, half), F32),
        ]

    @functools.partial(
        pl.kernel,
        out_type=jax.ShapeDtypeStruct((NPAD, width), F32),
        mesh=_mesh(),
        scratch_types=scratch,
    )
    def segsum(vals, rows, zeros, out, i0, v0, i1, v1, i2, v2, acc,
               si0, sv0, si1, sv1, si2, sv2, *tails):
        cid = lax.axis_index("c")
        sid = lax.axis_index("s")
        pltpu.sync_copy(
            zeros.at[pl.ds(sid * RPS, RPS), pl.ds(0, half)],
            acc.at[pl.ds(sid * RPS, RPS)],
        )
        plsc.subcore_barrier()

        base0 = sid * per
        pairs = ((i0, v0, si0, sv0), (i1, v1, si1, sv1), (i2, v2, si2, sv2))

        def issue(k, p):
            iv, vv, si, sv = p
            pltpu.async_copy(rows.at[pl.ds(base0 + k * CH, CH)], iv, si)
            pltpu.async_copy(
                vals.at[pl.ds(base0 + k * CH, CH), pl.ds(cid * half, half)],
                vv, sv)

        def wait(k, p):
            iv, vv, si, sv = p
            pltpu.make_async_copy(
                rows.at[pl.ds(base0 + k * CH, CH)], iv, si).wait()
            pltpu.make_async_copy(
                vals.at[pl.ds(base0 + k * CH, CH), pl.ds(cid * half, half)],
                vv, sv).wait()

        for b in range(3):
            issue(b, pairs[b])

        def group(g, c):
            for b in range(3):
                k = 3 * g + b
                p = pairs[b]
                wait(k, p)
                pltpu.sync_copy(p[1], acc.at[p[0]], add=True)
                issue(k + 3, p)
            return c

        lax.fori_loop(0, ngroups - 1, group, 0)
        for b in range(3):
            k = 3 * (ngroups - 1) + b
            p = pairs[b]
            wait(k, p)
            pltpu.sync_copy(p[1], acc.at[p[0]], add=True)
        if tail:
            ti, tv = tails
            kb = base0 + n_full * CH
            pltpu.sync_copy(rows.at[pl.ds(kb, tail)], ti)
            pltpu.sync_copy(
                vals.at[pl.ds(kb, tail), pl.ds(cid * half, half)], tv)
            pltpu.sync_copy(tv, acc.at[ti], add=True)

        plsc.subcore_barrier()
        pltpu.sync_copy(
            acc.at[pl.ds(sid * RPS, RPS)],
            out.at[pl.ds(sid * RPS, RPS), pl.ds(cid * half, half)],
        )

    return segsum


@functools.cache
def _make_segsum_part(n_edges):
    width = 128
    per_core = n_edges // NC
    per = per_core // NS
    CH = 64
    n_full = per // CH
    tail = per - n_full * CH
    ngroups = n_full // 3
    assert n_full % 3 == 0 and tail % 8 == 0

    scratch = [
        pltpu.VMEM((CH,), jnp.int32),
        pltpu.VMEM((CH, width), F32),
        pltpu.VMEM((CH,), jnp.int32),
        pltpu.VMEM((CH, width), F32),
        pltpu.VMEM((CH,), jnp.int32),
        pltpu.VMEM((CH, width), F32),
        pltpu.VMEM_SHARED((NPAD, width), F32),
    ] + [pltpu.SemaphoreType.DMA] * 6
    if tail:
        scratch += [
            pltpu.VMEM((tail,), jnp.int32),
            pltpu.VMEM((tail, width), F32),
        ]

    @functools.partial(
        pl.kernel,
        out_type=jax.ShapeDtypeStruct((NC, NPAD, width), F32),
        mesh=_mesh(),
        scratch_types=scratch,
    )
    def segsum_part(vals, rows, zeros, out, i0, v0, i1, v1, i2, v2, acc,
                    si0, sv0, si1, sv1, si2, sv2, *tails):
        cid = lax.axis_index("c")
        sid = lax.axis_index("s")
        pltpu.sync_copy(
            zeros.at[pl.ds(sid * RPS, RPS)],
            acc.at[pl.ds(sid * RPS, RPS)],
        )
        plsc.subcore_barrier()

        base0 = cid * per_core + sid * per
        pairs = ((i0, v0, si0, sv0), (i1, v1, si1, sv1), (i2, v2, si2, sv2))

        def issue(k, p):
            iv, vv, si, sv = p
            pltpu.async_copy(rows.at[pl.ds(base0 + k * CH, CH)], iv, si)
            pltpu.async_copy(vals.at[pl.ds(base0 + k * CH, CH)], vv, sv)

        def wait(k, p):
            iv, vv, si, sv = p
            pltpu.make_async_copy(
                rows.at[pl.ds(base0 + k * CH, CH)], iv, si).wait()
            pltpu.make_async_copy(
                vals.at[pl.ds(base0 + k * CH, CH)], vv, sv).wait()

        for b in range(3):
            issue(b, pairs[b])

        def group(g, c):
            for b in range(3):
                k = 3 * g + b
                p = pairs[b]
                wait(k, p)
                pltpu.sync_copy(p[1], acc.at[p[0]], add=True)
                issue(k + 3, p)
            return c

        lax.fori_loop(0, ngroups - 1, group, 0)
        for b in range(3):
            k = 3 * (ngroups - 1) + b
            p = pairs[b]
            wait(k, p)
            pltpu.sync_copy(p[1], acc.at[p[0]], add=True)
        if tail:
            ti, tv = tails
            kb = base0 + n_full * CH
            pltpu.sync_copy(rows.at[pl.ds(kb, tail)], ti)
            pltpu.sync_copy(vals.at[pl.ds(kb, tail)], tv)
            pltpu.sync_copy(tv, acc.at[ti], add=True)

        plsc.subcore_barrier()
        pltpu.sync_copy(
            acc.at[pl.ds(sid * RPS, RPS)],
            out.at[cid, pl.ds(sid * RPS, RPS)],
        )

    return segsum_part


def _silu(v):
    return v * jax.nn.sigmoid(v)


def _dot(a, b):
    return jnp.dot(a, b, preferred_element_type=F32)


BF16 = jnp.bfloat16


def _dotb(a, b):
    return jnp.dot(a.astype(BF16), b, preferred_element_type=F32)


def _bf16_word(lo_f32, hi_f32):
    ulo = lax.bitcast_convert_type(lo_f32, jnp.uint32)
    uhi = lax.bitcast_convert_type(hi_f32, jnp.uint32)
    rlo = (ulo + 0x7FFF + ((ulo >> 16) & 1)) >> 16
    rhi = (uhi + 0x7FFF + ((uhi >> 16) & 1)) >> 16
    return lax.bitcast_convert_type(rlo | (rhi << 16), jnp.int32)


def _unpack_words(w32):
    u = lax.bitcast_convert_type(w32, jnp.uint32)
    lo = lax.bitcast_convert_type(u << 16, F32)
    hi = lax.bitcast_convert_type(u & jnp.uint32(0xFFFF0000), F32)
    return lo, hi


_BN = 2000
_BE = 1600


def _full(shape):
    return pl.BlockSpec(shape, lambda i: (0,) * len(shape))


def _proj_body(h, wrlo, wrhi, wclo, wchi, hr, hc):
    hv = h[...]
    hr[...] = _bf16_word(_dotb(hv, wrlo[...]), _dotb(hv, wrhi[...]))
    hc[...] = _bf16_word(_dotb(hv, wclo[...]), _dotb(hv, wchi[...]))


def _proj(h, wrlo, wrhi, wclo, wchi):
    n, d = h.shape
    hw = d // 2
    return pl.pallas_call(
        _proj_body,
        grid=(n // _BN,),
        in_specs=[
            pl.BlockSpec((_BN, d), lambda i: (i, 0)),
            _full((d, hw)), _full((d, hw)),
            _full((d, hw)), _full((d, hw)),
        ],
        out_specs=[pl.BlockSpec((_BN, hw), lambda i: (i, 0))] * 2,
        out_shape=[jax.ShapeDtypeStruct((n, hw), jnp.int32)] * 2,
    )(h, wrlo, wrhi, wclo, wchi)


def _edge_first_body(gr, gc, diff, ea, wea, wrad, be1, we2, be2,
                     out, rad8):
    d = diff[...][:, :16]
    radial = jnp.sum(d * d, axis=1, keepdims=True)
    rad8[...] = jnp.broadcast_to(radial, (radial.shape[0], 8))
    glo, ghi = _unpack_words(gr[...])
    clo, chi = _unpack_words(gc[...])
    msum = jnp.concatenate([glo + clo, ghi + chi], axis=1)
    pre = (
        msum + _dot(ea[...], wea[...])
        + radial * wrad[...] + be1[...]
    )
    m = _silu(pre)
    out[...] = _silu(_dotb(m, we2[...]) + be2[...])


def _edge_mid_body(gr, gc, rad, ea, wea, wrad, be1, we2, be2, out):
    radial = rad[...][:, :1]
    glo, ghi = _unpack_words(gr[...])
    clo, chi = _unpack_words(gc[...])
    msum = jnp.concatenate([glo + clo, ghi + chi], axis=1)
    pre = (
        msum + _dot(ea[...], wea[...])
        + radial * wrad[...] + be1[...]
    )
    m = _silu(pre)
    out[...] = _silu(_dotb(m, we2[...]) + be2[...])


def _edge_last_body(gr, gc, diff, ea, wea, wrad, be1, we2, be2,
                    wc1, bc1, wc2t, out, trans):
    d = diff[...][:, :16]
    radial = jnp.sum(d * d, axis=1, keepdims=True)
    glo, ghi = _unpack_words(gr[...])
    clo, chi = _unpack_words(gc[...])
    msum = jnp.concatenate([glo + clo, ghi + chi], axis=1)
    pre = (
        msum + _dot(ea[...], wea[...])
        + radial * wrad[...] + be1[...]
    )
    m = _silu(pre)
    m2 = _silu(_dotb(m, we2[...]) + be2[...])
    out[...] = m2
    c1 = _silu(_dotb(m2, wc1[...]) + bc1[...])
    w = jnp.sum(c1 * wc2t[...], axis=1, keepdims=True)
    trans[...] = jnp.concatenate(
        [d * w, jnp.zeros((d.shape[0], 112), F32)], axis=1
    )


def _edge_mlp(gr, gc, aux, ea, wea, wrad, be1, we2, be2,
              mode="mid", coord=None, be=_BE):
    e = gr.shape[0]
    d = gr.shape[1] * 2
    de = ea.shape[1]
    edge_spec = pl.BlockSpec((be, d), lambda i: (i, 0))
    word_spec = pl.BlockSpec((be, d // 2), lambda i: (i, 0))
    diff_spec = pl.BlockSpec((be, 128), lambda i: (i, 0))
    rad_spec = pl.BlockSpec((be, 8), lambda i: (i, 0))
    ea_spec = pl.BlockSpec((be, de), lambda i: (i, 0))
    aux_spec = rad_spec if mode == "mid" else diff_spec
    in_specs = [
        word_spec, word_spec, aux_spec, ea_spec,
        _full((de, d)), _full((1, d)), _full((1, d)),
        _full((d, d)), _full((1, d)),
    ]
    args = [gr, gc, aux, ea, wea, wrad, be1, we2, be2]
    if mode == "first":
        return pl.pallas_call(
            _edge_first_body,
            grid=(e // be,),
            in_specs=in_specs,
            out_specs=[edge_spec, rad_spec],
            out_shape=[
                jax.ShapeDtypeStruct((e, d), F32),
                jax.ShapeDtypeStruct((e, 8), F32),
            ],
        )(*args)
    if mode == "mid":
        return pl.pallas_call(
            _edge_mid_body,
            grid=(e // be,),
            in_specs=in_specs,
            out_specs=edge_spec,
            out_shape=jax.ShapeDtypeStruct((e, d), F32),
        )(*args)
    wc1, bc1, wc2t = coord
    return pl.pallas_call(
        _edge_last_body,
        grid=(e // be,),
        in_specs=in_specs + [_full((d, d)), _full((1, d)), _full((1, d))],
        out_specs=[edge_spec, pl.BlockSpec((be, 128), lambda i: (i, 0))],
        out_shape=[
            jax.ShapeDtypeStruct((e, d), F32),
            jax.ShapeDtypeStruct((e, 128), F32),
        ],
    )(*args, wc1, bc1, wc2t)


def _node_body(h, agga, aggb, wn1h, wn1a, bn1, wn2, bn2,
               wrlo, wrhi, wclo, wchi, out_h, out_hr, out_hc):
    hv = h[...]
    agg = agga[...] + aggb[...]
    t = _silu(_dotb(hv, wn1h[...]) + _dotb(agg, wn1a[...]) + bn1[...])
    hn = hv + _dotb(t, wn2[...]) + bn2[...]
    out_h[...] = hn
    out_hr[...] = _bf16_word(_dotb(hn, wrlo[...]), _dotb(hn, wrhi[...]))
    out_hc[...] = _bf16_word(_dotb(hn, wclo[...]), _dotb(hn, wchi[...]))


def _node_mlp(h, agga, aggb, wn1h, wn1a, bn1, wn2, bn2,
              wrlo, wrhi, wclo, wchi):
    n, d = h.shape
    hw = d // 2
    node_spec = pl.BlockSpec((_BN, d), lambda i: (i, 0))
    word_spec = pl.BlockSpec((_BN, hw), lambda i: (i, 0))
    return pl.pallas_call(
        _node_body,
        grid=(n // _BN,),
        in_specs=[
            node_spec, node_spec, node_spec,
            _full((d, d)), _full((d, d)), _full((1, d)),
            _full((d, d)), _full((1, d)),
            _full((d, hw)), _full((d, hw)),
            _full((d, hw)), _full((d, hw)),
        ],
        out_specs=[node_spec, word_spec, word_spec],
        out_shape=[jax.ShapeDtypeStruct((n, d), F32),
                   jax.ShapeDtypeStruct((n, hw), jnp.int32),
                   jax.ShapeDtypeStruct((n, hw), jnp.int32)],
    )(h, agga, aggb, wn1h, wn1a, bn1, wn2, bn2,
      wrlo, wrhi, wclo, wchi)


def _node_last_body(h, agga, aggb, x16, p0, p1, p2, p3,
                    wn1h, wn1a, bn1, wn2, bn2, out_h, out_x):
    hv = h[...]
    agg = agga[...] + aggb[...]
    t = _silu(_dotb(hv, wn1h[...]) + _dotb(agg, wn1a[...]) + bn1[...])
    out_h[...] = hv + _dotb(t, wn2[...]) + bn2[...]
    out_x[...] = x16[...] + p0[...] + p1[...] + p2[...] + p3[...]


def _node_mlp_last(h, agga, aggb, x16, parts,
                   wn1h, wn1a, bn1, wn2, bn2):
    n, d = h.shape
    node_spec = pl.BlockSpec((_BN, d), lambda i: (i, 0))
    nar_spec = pl.BlockSpec((_BN, 16), lambda i: (i, 0))
    return pl.pallas_call(
        _node_last_body,
        grid=(n // _BN,),
        in_specs=[
            node_spec, node_spec, node_spec,
            nar_spec, nar_spec, nar_spec, nar_spec, nar_spec,
            _full((d, d)), _full((d, d)), _full((1, d)),
            _full((d, d)), _full((1, d)),
        ],
        out_specs=[node_spec, nar_spec],
        out_shape=[
            jax.ShapeDtypeStruct((n, d), F32),
            jax.ShapeDtypeStruct((n, 16), F32),
        ],
    )(h, agga, aggb, x16, *parts, wn1h, wn1a, bn1, wn2, bn2)


def kernel(h, x, edges, edge_attr, params):
    layers = params["layers"]
    n, d = h.shape
    e = edges.shape[1]
    de = edge_attr.shape[1]
    row = edges[0]
    col = edges[1]

    def split(p, with_coord):
        we1 = p["We1"]
        out = {
            "wrlo": we1[:d, :d // 2].astype(BF16),
            "wrhi": we1[:d, d // 2:].astype(BF16),
            "wclo": we1[d:2 * d, :d // 2].astype(BF16),
            "wchi": we1[d:2 * d, d // 2:].astype(BF16),
            "wrad": we1[2 * d:2 * d + 1],
            "wea": we1[2 * d + 1:],
            "be1": p["be1"].reshape(1, d),
            "we2": p["We2"].astype(BF16),
            "be2": p["be2"].reshape(1, d),
            "wn1h": p["Wn1"][:d].astype(BF16),
            "wn1a": p["Wn1"][d:].astype(BF16),
            "bn1": p["bn1"].reshape(1, d),
            "wn2": p["Wn2"].astype(BF16),
            "bn2": p["bn2"].reshape(1, d),
        }
        if with_coord:
            out["wc1"] = p["Wc1"].astype(BF16)
            out["bc1"] = p["bc1"].reshape(1, d)
            out["wc2t"] = p["Wc2"].reshape(1, d)
        return out

    nl = len(layers)
    ps = [split(p, i == nl - 1) for i, p in enumerate(layers)]

    x16 = jnp.pad(x, ((0, 0), (0, 16 - x.shape[1])))
    x128 = jnp.pad(x, ((0, 0), (0, 128 - x.shape[1])))
    zeros128 = jnp.zeros((NPAD, 128), F32)

    ea_n = 79872
    eb_n = e - ea_n
    bes = {ea_n: 1664, eb_n: 2504}
    halves = (
        (row[:ea_n], col[:ea_n], edge_attr[:ea_n]),
        (row[ea_n:], col[ea_n:], edge_attr[ea_n:]),
    )
    gd = {}
    g2 = {}
    seg = {}
    segp = {}
    for en in (ea_n, eb_n):
        gd[en] = _make_gather2sum(n, 128, en, True)
        g2[en] = _make_gather2(n, d // 2, en, jnp.int32)
        seg[en] = _make_segsum(en, d)
        segp[en] = _make_segsum_part(en)

    diffs = [gd[r.shape[0]](x128, x128, r, c) for r, c, _ in halves]

    hr, hc = _proj(h, ps[0]["wrlo"], ps[0]["wrhi"],
                   ps[0]["wclo"], ps[0]["wchi"])
    rads = [None, None]
    for i, p in enumerate(ps):
        gath = [g2[r.shape[0]](hr, hc, r, c) for r, c, _ in halves]
        m2s = []
        transs = []
        for j, (r, c, ea_h) in enumerate(halves):
            en = r.shape[0]
            if i == 0:
                m2, rads[j] = _edge_mlp(gath[j][0], gath[j][1], diffs[j],
                                        ea_h, p["wea"], p["wrad"], p["be1"],
                                        p["we2"], p["be2"], mode="first",
                                        be=bes[en])
            elif i < nl - 1:
                m2 = _edge_mlp(gath[j][0], gath[j][1], rads[j],
                               ea_h, p["wea"], p["wrad"], p["be1"],
                               p["we2"], p["be2"], mode="mid", be=bes[en])
            else:
                m2, trans = _edge_mlp(gath[j][0], gath[j][1], diffs[j],
                                      ea_h, p["wea"], p["wrad"], p["be1"],
                                      p["we2"], p["be2"], mode="last",
                                      coord=(p["wc1"], p["bc1"], p["wc2t"]),
                                      be=bes[en])
                transs.append(trans)
            m2s.append(m2)
        aggs = [seg[r.shape[0]](m2s[j], r, zeros128)
                for j, (r, c, _) in enumerate(halves)]
        if i < nl - 1:
            pn = ps[i + 1]
            h, hr, hc = _node_mlp(h, aggs[0], aggs[1],
                                  p["wn1h"], p["wn1a"], p["bn1"],
                                  p["wn2"], p["bn2"],
                                  pn["wrlo"], pn["wrhi"],
                                  pn["wclo"], pn["wchi"])
        else:
            parts = [segp[r.shape[0]](transs[j], r, zeros128)
                     for j, (r, c, _) in enumerate(halves)]
            pl4 = [parts[0][0, :n, :16], parts[0][1, :n, :16],
                   parts[1][0, :n, :16], parts[1][1, :n, :16]]
            h, x16o = _node_mlp_last(h, aggs[0], aggs[1], x16, pl4,
                                     p["wn1h"], p["wn1a"], p["bn1"],
                                     p["wn2"], p["bn2"])
    return (h, x16o[:, :3])


# speedup vs baseline: 4.1373x; 1.0004x over previous
import functools

import jax
import jax.numpy as jnp
from jax import lax
from jax.experimental import pallas as pl
from jax.experimental.pallas import tpu as pltpu
from jax.experimental.pallas import tpu_sc as plsc

F32 = jnp.float32

NC = 2
NS = 16
NW = NC * NS
NPAD = 10240
RPS = NPAD // NS


@functools.cache
def _mesh():
    return plsc.VectorSubcoreMesh(
        core_axis_name="c", subcore_axis_name="s",
        num_cores=NC, num_subcores=NS,
    )


@functools.cache
def _make_gather2sum(n_rows, width, n_edges, subtract, dtype=F32):
    per = n_edges // NW
    assert per * NW == n_edges
    CH = 64
    n_full = per // CH
    tail = per - n_full * CH
    ngroups = n_full // 3
    assert n_full % 3 == 0 and tail % 8 == 0
    isbf = dtype == jnp.bfloat16
    wdt = jnp.int32 if isbf else F32
    nslice = width // 16

    scratch = [
        pltpu.VMEM((per,), jnp.int32),
        pltpu.VMEM((per,), jnp.int32),
    ]
    for _ in range(3):
        scratch += [pltpu.VMEM((CH, width), wdt)] * 3
    scratch += [pltpu.SemaphoreType.DMA] * 6
    if tail:
        scratch += [pltpu.VMEM((tail, width), wdt)] * 3

    @functools.partial(
        pl.kernel,
        out_type=jax.ShapeDtypeStruct((n_edges, width), wdt),
        mesh=_mesh(),
        scratch_types=scratch,
    )
    def gather2sum(ta, tb, ia, ib, out, idxa, idxb,
                   a0, b0, o0, a1, b1, o1, a2, b2, o2,
                   sa0, sb0, sa1, sb1, sa2, sb2, *tails):
        wid = lax.axis_index("s") * NC + lax.axis_index("c")
        base0 = wid * per
        pltpu.sync_copy(ia.at[pl.ds(base0, per)], idxa)
        pltpu.sync_copy(ib.at[pl.ds(base0, per)], idxb)
        pairs = ((a0, b0, o0, sa0, sb0), (a1, b1, o1, sa1, sb1),
                 (a2, b2, o2, sa2, sb2))

        def issue(k, p):
            ba, bb, _, sa, sb = p
            pltpu.async_copy(ta.at[idxa.at[pl.ds(k * CH, CH)]], ba, sa)
            pltpu.async_copy(tb.at[idxb.at[pl.ds(k * CH, CH)]], bb, sb)

        def wait(k, p):
            ba, bb, _, sa, sb = p
            pltpu.make_async_copy(
                ta.at[idxa.at[pl.ds(k * CH, CH)]], ba, sa).wait()
            pltpu.make_async_copy(
                tb.at[idxb.at[pl.ds(k * CH, CH)]], bb, sb).wait()

        def combine(ba, bb, bo, n):
            def rowbody(r, c):
                for j in range(nslice):
                    sl = (r, pl.ds(j * 16, 16))
                    if isbf:
                        va = plsc.bitcast(ba[sl], jnp.bfloat16)
                        vb = plsc.bitcast(bb[sl], jnp.bfloat16)
                        v = va - vb if subtract else va + vb
                        bo[sl] = plsc.bitcast(v, jnp.int32)
                    else:
                        v = ba[sl] - bb[sl] if subtract else ba[sl] + bb[sl]
                        bo[sl] = v
                return c
            lax.fori_loop(0, n, rowbody, 0)

        for b in range(3):
            issue(b, pairs[b])

        def group(g, c):
            for b in range(3):
                k = 3 * g + b
                p = pairs[b]
                wait(k, p)
                combine(p[0], p[1], p[2], CH)
                issue(k + 3, p)
                pltpu.sync_copy(p[2], out.at[pl.ds(base0 + k * CH, CH)])
            return c

        lax.fori_loop(0, ngroups - 1, group, 0)
        for b in range(3):
            k = 3 * (ngroups - 1) + b
            p = pairs[b]
            wait(k, p)
            combine(p[0], p[1], p[2], CH)
            pltpu.sync_copy(p[2], out.at[pl.ds(base0 + k * CH, CH)])
        if tail:
            tba, tbb, tbo = tails
            kb = n_full * CH
            pltpu.async_copy(
                ta.at[idxa.at[pl.ds(kb, tail)]], tba, sa0).wait()
            pltpu.async_copy(
                tb.at[idxb.at[pl.ds(kb, tail)]], tbb, sb0).wait()
            combine(tba, tbb, tbo, tail)
            pltpu.sync_copy(tbo, out.at[pl.ds(base0 + kb, tail)])

    return gather2sum


@functools.cache
def _make_gather2(n_rows, width, n_edges, wdt):
    per = n_edges // NW
    assert per * NW == n_edges
    CH = 64
    n_full = per // CH
    tail = per - n_full * CH
    ngroups = n_full // 3
    assert n_full % 3 == 0 and tail % 8 == 0
    scratch = [
        pltpu.VMEM((per,), jnp.int32),
        pltpu.VMEM((per,), jnp.int32),
    ]
    for _ in range(3):
        scratch += [pltpu.VMEM((CH, width), wdt)] * 2
    scratch += [pltpu.SemaphoreType.DMA] * 6
    if tail:
        scratch += [pltpu.VMEM((tail, width), wdt)] * 2

    @functools.partial(
        pl.kernel,
        out_type=(
            jax.ShapeDtypeStruct((n_edges, width), wdt),
            jax.ShapeDtypeStruct((n_edges, width), wdt),
        ),
        mesh=_mesh(),
        scratch_types=scratch,
    )
    def gather2(ta, tb, ia, ib, oa, ob, idxa, idxb,
                a0, b0, a1, b1, a2, b2,
                sa0, sb0, sa1, sb1, sa2, sb2, *tails):
        wid = lax.axis_index("s") * NC + lax.axis_index("c")
        base0 = wid * per
        pltpu.sync_copy(ia.at[pl.ds(base0, per)], idxa)
        pltpu.sync_copy(ib.at[pl.ds(base0, per)], idxb)
        pairs = ((a0, b0, sa0, sb0), (a1, b1, sa1, sb1), (a2, b2, sa2, sb2))

        def issue(k, p):
            ba, bb, sa, sb = p
            pltpu.async_copy(ta.at[idxa.at[pl.ds(k * CH, CH)]], ba, sa)
            pltpu.async_copy(tb.at[idxb.at[pl.ds(k * CH, CH)]], bb, sb)

        def wait(k, p):
            ba, bb, sa, sb = p
            pltpu.make_async_copy(
                ta.at[idxa.at[pl.ds(k * CH, CH)]], ba, sa).wait()
            pltpu.make_async_copy(
                tb.at[idxb.at[pl.ds(k * CH, CH)]], bb, sb).wait()

        def drain(k, p):
            ba, bb, sa, sb = p
            pltpu.sync_copy(ba, oa.at[pl.ds(base0 + k * CH, CH)])
            pltpu.sync_copy(bb, ob.at[pl.ds(base0 + k * CH, CH)])

        for b in range(3):
            issue(b, pairs[b])

        def group(g, c):
            for b in range(3):
                k = 3 * g + b
                p = pairs[b]
                wait(k, p)
                drain(k, p)
                issue(k + 3, p)
            return c

        lax.fori_loop(0, ngroups - 1, group, 0)
        for b in range(3):
            k = 3 * (ngroups - 1) + b
            p = pairs[b]
            wait(k, p)
            drain(k, p)
        if tail:
            tba, tbb = tails
            kb = n_full * CH
            pltpu.async_copy(
                ta.at[idxa.at[pl.ds(kb, tail)]], tba, sa0).wait()
            pltpu.async_copy(
                tb.at[idxb.at[pl.ds(kb, tail)]], tbb, sb0).wait()
            pltpu.sync_copy(tba, oa.at[pl.ds(base0 + kb, tail)])
            pltpu.sync_copy(tbb, ob.at[pl.ds(base0 + kb, tail)])

    return gather2


@functools.cache
def _make_segsum(n_edges, width):
    half = width // NC
    per = n_edges // NS
    assert per * NS == n_edges
    CH = 104
    n_full = per // CH
    tail = per - n_full * CH
    ngroups = n_full // 3
    assert n_full % 3 == 0 and tail % 8 == 0

    scratch = [
        pltpu.VMEM((CH,), jnp.int32),
        pltpu.VMEM((CH, half), F32),
        pltpu.VMEM((CH,), jnp.int32),
        pltpu.VMEM((CH, half), F32),
        pltpu.VMEM((CH,), jnp.int32),
        pltpu.VMEM((CH, half), F32),
        pltpu.VMEM_SHARED((NPAD, half), F32),
    ] + [pltpu.SemaphoreType.DMA] * 6
    if tail:
        scratch += [
            pltpu.VMEM((tail,), jnp.int32),
            pltpu.VMEM((tail, half), F32),
        ]

    @functools.partial(
        pl.kernel,
        out_type=jax.ShapeDtypeStruct((NPAD, width), F32),
        mesh=_mesh(),
        scratch_types=scratch,
    )
    def segsum(vals, rows, zeros, out, i0, v0, i1, v1, i2, v2, acc,
               si0, sv0, si1, sv1, si2, sv2, *tails):
        cid = lax.axis_index("c")
        sid = lax.axis_index("s")
        pltpu.sync_copy(
            zeros.at[pl.ds(sid * RPS, RPS), pl.ds(0, half)],
            acc.at[pl.ds(sid * RPS, RPS)],
        )
        plsc.subcore_barrier()

        base0 = sid * per
        pairs = ((i0, v0, si0, sv0), (i1, v1, si1, sv1), (i2, v2, si2, sv2))

        def issue(k, p):
            iv, vv, si, sv = p
            pltpu.async_copy(rows.at[pl.ds(base0 + k * CH, CH)], iv, si)
            pltpu.async_copy(
                vals.at[pl.ds(base0 + k * CH, CH), pl.ds(cid * half, half)],
                vv, sv)

        def wait(k, p):
            iv, vv, si, sv = p
            pltpu.make_async_copy(
                rows.at[pl.ds(base0 + k * CH, CH)], iv, si).wait()
            pltpu.make_async_copy(
                vals.at[pl.ds(base0 + k * CH, CH), pl.ds(cid * half, half)],
                vv, sv).wait()

        for b in range(3):
            issue(b, pairs[b])

        def group(g, c):
            for b in range(3):
                k = 3 * g + b
                p = pairs[b]
                wait(k, p)
                pltpu.sync_copy(p[1], acc.at[p[0]], add=True)
                issue(k + 3, p)
            return c

        lax.fori_loop(0, ngroups - 1, group, 0)
        for b in range(3):
            k = 3 * (ngroups - 1) + b
            p = pairs[b]
            wait(k, p)
            pltpu.sync_copy(p[1], acc.at[p[0]], add=True)
        if tail:
            ti, tv = tails
            kb = base0 + n_full * CH
            pltpu.sync_copy(rows.at[pl.ds(kb, tail)], ti)
            pltpu.sync_copy(
                vals.at[pl.ds(kb, tail), pl.ds(cid * half, half)], tv)
            pltpu.sync_copy(tv, acc.at[ti], add=True)

        plsc.subcore_barrier()
        pltpu.sync_copy(
            acc.at[pl.ds(sid * RPS, RPS)],
            out.at[pl.ds(sid * RPS, RPS), pl.ds(cid * half, half)],
        )

    return segsum


@functools.cache
def _make_segsum_part(n_edges):
    width = 128
    per_core = n_edges // NC
    per = per_core // NS
    CH = 64
    n_full = per // CH
    tail = per - n_full * CH
    ngroups = n_full // 3
    assert n_full % 3 == 0 and tail % 8 == 0

    scratch = [
        pltpu.VMEM((CH,), jnp.int32),
        pltpu.VMEM((CH, width), F32),
        pltpu.VMEM((CH,), jnp.int32),
        pltpu.VMEM((CH, width), F32),
        pltpu.VMEM((CH,), jnp.int32),
        pltpu.VMEM((CH, width), F32),
        pltpu.VMEM_SHARED((NPAD, width), F32),
    ] + [pltpu.SemaphoreType.DMA] * 6
    if tail:
        scratch += [
            pltpu.VMEM((tail,), jnp.int32),
            pltpu.VMEM((tail, width), F32),
        ]

    @functools.partial(
        pl.kernel,
        out_type=jax.ShapeDtypeStruct((NC, NPAD, width), F32),
        mesh=_mesh(),
        scratch_types=scratch,
    )
    def segsum_part(vals, rows, zeros, out, i0, v0, i1, v1, i2, v2, acc,
                    si0, sv0, si1, sv1, si2, sv2, *tails):
        cid = lax.axis_index("c")
        sid = lax.axis_index("s")
        pltpu.sync_copy(
            zeros.at[pl.ds(sid * RPS, RPS)],
            acc.at[pl.ds(sid * RPS, RPS)],
        )
        plsc.subcore_barrier()

        base0 = cid * per_core + sid * per
        pairs = ((i0, v0, si0, sv0), (i1, v1, si1, sv1), (i2, v2, si2, sv2))

        def issue(k, p):
            iv, vv, si, sv = p
            pltpu.async_copy(rows.at[pl.ds(base0 + k * CH, CH)], iv, si)
            pltpu.async_copy(vals.at[pl.ds(base0 + k * CH, CH)], vv, sv)

        def wait(k, p):
            iv, vv, si, sv = p
            pltpu.make_async_copy(
                rows.at[pl.ds(base0 + k * CH, CH)], iv, si).wait()
            pltpu.make_async_copy(
                vals.at[pl.ds(base0 + k * CH, CH)], vv, sv).wait()

        for b in range(3):
            issue(b, pairs[b])

        def group(g, c):
            for b in range(3):
                k = 3 * g + b
                p = pairs[b]
                wait(k, p)
                pltpu.sync_copy(p[1], acc.at[p[0]], add=True)
                issue(k + 3, p)
            return c

        lax.fori_loop(0, ngroups - 1, group, 0)
        for b in range(3):
            k = 3 * (ngroups - 1) + b
            p = pairs[b]
            wait(k, p)
            pltpu.sync_copy(p[1], acc.at[p[0]], add=True)
        if tail:
            ti, tv = tails
            kb = base0 + n_full * CH
            pltpu.sync_copy(rows.at[pl.ds(kb, tail)], ti)
            pltpu.sync_copy(vals.at[pl.ds(kb, tail)], tv)
            pltpu.sync_copy(tv, acc.at[ti], add=True)

        plsc.subcore_barrier()
        pltpu.sync_copy(
            acc.at[pl.ds(sid * RPS, RPS)],
            out.at[cid, pl.ds(sid * RPS, RPS)],
        )

    return segsum_part


def _silu(v):
    return v * (0.5 * jnp.tanh(v * 0.5) + 0.5)


def _dot(a, b):
    return jnp.dot(a, b, preferred_element_type=F32)


BF16 = jnp.bfloat16


def _dotb(a, b):
    return jnp.dot(a.astype(BF16), b, preferred_element_type=F32)


def _bf16_word(lo_f32, hi_f32):
    ulo = lax.bitcast_convert_type(lo_f32, jnp.uint32)
    uhi = lax.bitcast_convert_type(hi_f32, jnp.uint32)
    rlo = (ulo + 0x7FFF + ((ulo >> 16) & 1)) >> 16
    rhi = (uhi + 0x7FFF + ((uhi >> 16) & 1)) >> 16
    return lax.bitcast_convert_type(rlo | (rhi << 16), jnp.int32)


def _unpack_words(w32):
    u = lax.bitcast_convert_type(w32, jnp.uint32)
    lo = lax.bitcast_convert_type(u << 16, F32)
    hi = lax.bitcast_convert_type(u & jnp.uint32(0xFFFF0000), F32)
    return lo, hi


_BN = 2000
_BE = 1600


def _full(shape):
    return pl.BlockSpec(shape, lambda i: (0,) * len(shape))


def _proj_body(h, wrlo, wrhi, wclo, wchi, hr, hc):
    hv = h[...]
    hr[...] = _bf16_word(_dotb(hv, wrlo[...]), _dotb(hv, wrhi[...]))
    hc[...] = _bf16_word(_dotb(hv, wclo[...]), _dotb(hv, wchi[...]))


def _proj(h, wrlo, wrhi, wclo, wchi):
    n, d = h.shape
    hw = d // 2
    return pl.pallas_call(
        _proj_body,
        grid=(n // _BN,),
        in_specs=[
            pl.BlockSpec((_BN, d), lambda i: (i, 0)),
            _full((d, hw)), _full((d, hw)),
            _full((d, hw)), _full((d, hw)),
        ],
        out_specs=[pl.BlockSpec((_BN, hw), lambda i: (i, 0))] * 2,
        out_shape=[jax.ShapeDtypeStruct((n, hw), jnp.int32)] * 2,
    )(h, wrlo, wrhi, wclo, wchi)


def _edge_first_body(gr, gc, diff, ea, wea, wrad, be1, we2, be2,
                     out, rad8):
    d = diff[...][:, :16]
    radial = jnp.sum(d * d, axis=1, keepdims=True)
    rad8[...] = jnp.broadcast_to(radial, (radial.shape[0], 8))
    glo, ghi = _unpack_words(gr[...])
    clo, chi = _unpack_words(gc[...])
    msum = jnp.concatenate([glo + clo, ghi + chi], axis=1)
    pre = (
        msum + _dot(ea[...], wea[...])
        + radial * wrad[...] + be1[...]
    )
    m = _silu(pre)
    out[...] = _silu(_dotb(m, we2[...]) + be2[...])


def _edge_mid_body(gr, gc, rad, ea, wea, wrad, be1, we2, be2, out):
    radial = rad[...][:, :1]
    glo, ghi = _unpack_words(gr[...])
    clo, chi = _unpack_words(gc[...])
    msum = jnp.concatenate([glo + clo, ghi + chi], axis=1)
    pre = (
        msum + _dot(ea[...], wea[...])
        + radial * wrad[...] + be1[...]
    )
    m = _silu(pre)
    out[...] = _silu(_dotb(m, we2[...]) + be2[...])


def _edge_last_body(gr, gc, diff, ea, wea, wrad, be1, we2, be2,
                    wc1, bc1, wc2t, out, trans):
    d = diff[...][:, :16]
    radial = jnp.sum(d * d, axis=1, keepdims=True)
    glo, ghi = _unpack_words(gr[...])
    clo, chi = _unpack_words(gc[...])
    msum = jnp.concatenate([glo + clo, ghi + chi], axis=1)
    pre = (
        msum + _dot(ea[...], wea[...])
        + radial * wrad[...] + be1[...]
    )
    m = _silu(pre)
    m2 = _silu(_dotb(m, we2[...]) + be2[...])
    out[...] = m2
    c1 = _silu(_dotb(m2, wc1[...]) + bc1[...])
    w = jnp.sum(c1 * wc2t[...], axis=1, keepdims=True)
    trans[...] = jnp.concatenate(
        [d * w, jnp.zeros((d.shape[0], 112), F32)], axis=1
    )


def _edge_mlp(gr, gc, aux, ea, wea, wrad, be1, we2, be2,
              mode="mid", coord=None, be=_BE):
    e = gr.shape[0]
    d = gr.shape[1] * 2
    de = ea.shape[1]
    edge_spec = pl.BlockSpec((be, d), lambda i: (i, 0))
    word_spec = pl.BlockSpec((be, d // 2), lambda i: (i, 0))
    diff_spec = pl.BlockSpec((be, 128), lambda i: (i, 0))
    rad_spec = pl.BlockSpec((be, 8), lambda i: (i, 0))
    ea_spec = pl.BlockSpec((be, de), lambda i: (i, 0))
    aux_spec = rad_spec if mode == "mid" else diff_spec
    in_specs = [
        word_spec, word_spec, aux_spec, ea_spec,
        _full((de, d)), _full((1, d)), _full((1, d)),
        _full((d, d)), _full((1, d)),
    ]
    args = [gr, gc, aux, ea, wea, wrad, be1, we2, be2]
    if mode == "first":
        return pl.pallas_call(
            _edge_first_body,
            grid=(e // be,),
            in_specs=in_specs,
            out_specs=[edge_spec, rad_spec],
            out_shape=[
                jax.ShapeDtypeStruct((e, d), F32),
                jax.ShapeDtypeStruct((e, 8), F32),
            ],
        )(*args)
    if mode == "mid":
        return pl.pallas_call(
            _edge_mid_body,
            grid=(e // be,),
            in_specs=in_specs,
            out_specs=edge_spec,
            out_shape=jax.ShapeDtypeStruct((e, d), F32),
        )(*args)
    wc1, bc1, wc2t = coord
    return pl.pallas_call(
        _edge_last_body,
        grid=(e // be,),
        in_specs=in_specs + [_full((d, d)), _full((1, d)), _full((1, d))],
        out_specs=[edge_spec, pl.BlockSpec((be, 128), lambda i: (i, 0))],
        out_shape=[
            jax.ShapeDtypeStruct((e, d), F32),
            jax.ShapeDtypeStruct((e, 128), F32),
        ],
    )(*args, wc1, bc1, wc2t)


def _node_body(h, agga, aggb, wn1h, wn1a, bn1, wn2, bn2,
               wrlo, wrhi, wclo, wchi, out_h, out_hr, out_hc):
    hv = h[...]
    agg = agga[...] + aggb[...]
    t = _silu(_dotb(hv, wn1h[...]) + _dotb(agg, wn1a[...]) + bn1[...])
    hn = hv + _dotb(t, wn2[...]) + bn2[...]
    out_h[...] = hn
    out_hr[...] = _bf16_word(_dotb(hn, wrlo[...]), _dotb(hn, wrhi[...]))
    out_hc[...] = _bf16_word(_dotb(hn, wclo[...]), _dotb(hn, wchi[...]))


def _node_mlp(h, agga, aggb, wn1h, wn1a, bn1, wn2, bn2,
              wrlo, wrhi, wclo, wchi):
    n, d = h.shape
    hw = d // 2
    node_spec = pl.BlockSpec((_BN, d), lambda i: (i, 0))
    word_spec = pl.BlockSpec((_BN, hw), lambda i: (i, 0))
    return pl.pallas_call(
        _node_body,
        grid=(n // _BN,),
        in_specs=[
            node_spec, node_spec, node_spec,
            _full((d, d)), _full((d, d)), _full((1, d)),
            _full((d, d)), _full((1, d)),
            _full((d, hw)), _full((d, hw)),
            _full((d, hw)), _full((d, hw)),
        ],
        out_specs=[node_spec, word_spec, word_spec],
        out_shape=[jax.ShapeDtypeStruct((n, d), F32),
                   jax.ShapeDtypeStruct((n, hw), jnp.int32),
                   jax.ShapeDtypeStruct((n, hw), jnp.int32)],
    )(h, agga, aggb, wn1h, wn1a, bn1, wn2, bn2,
      wrlo, wrhi, wclo, wchi)


def _node_last_body(h, agga, aggb, x16, p0, p1, p2, p3,
                    wn1h, wn1a, bn1, wn2, bn2, out_h, out_x):
    hv = h[...]
    agg = agga[...] + aggb[...]
    t = _silu(_dotb(hv, wn1h[...]) + _dotb(agg, wn1a[...]) + bn1[...])
    out_h[...] = hv + _dotb(t, wn2[...]) + bn2[...]
    out_x[...] = x16[...] + p0[...] + p1[...] + p2[...] + p3[...]


def _node_mlp_last(h, agga, aggb, x16, parts,
                   wn1h, wn1a, bn1, wn2, bn2):
    n, d = h.shape
    node_spec = pl.BlockSpec((_BN, d), lambda i: (i, 0))
    nar_spec = pl.BlockSpec((_BN, 16), lambda i: (i, 0))
    return pl.pallas_call(
        _node_last_body,
        grid=(n // _BN,),
        in_specs=[
            node_spec, node_spec, node_spec,
            nar_spec, nar_spec, nar_spec, nar_spec, nar_spec,
            _full((d, d)), _full((d, d)), _full((1, d)),
            _full((d, d)), _full((1, d)),
        ],
        out_specs=[node_spec, nar_spec],
        out_shape=[
            jax.ShapeDtypeStruct((n, d), F32),
            jax.ShapeDtypeStruct((n, 16), F32),
        ],
    )(h, agga, aggb, x16, *parts, wn1h, wn1a, bn1, wn2, bn2)


def kernel(h, x, edges, edge_attr, params):
    layers = params["layers"]
    n, d = h.shape
    e = edges.shape[1]
    de = edge_attr.shape[1]
    row = edges[0]
    col = edges[1]

    def split(p, with_coord):
        we1 = p["We1"]
        out = {
            "wrlo": we1[:d, :d // 2].astype(BF16),
            "wrhi": we1[:d, d // 2:].astype(BF16),
            "wclo": we1[d:2 * d, :d // 2].astype(BF16),
            "wchi": we1[d:2 * d, d // 2:].astype(BF16),
            "wrad": we1[2 * d:2 * d + 1],
            "wea": we1[2 * d + 1:],
            "be1": p["be1"].reshape(1, d),
            "we2": p["We2"].astype(BF16),
            "be2": p["be2"].reshape(1, d),
            "wn1h": p["Wn1"][:d].astype(BF16),
            "wn1a": p["Wn1"][d:].astype(BF16),
            "bn1": p["bn1"].reshape(1, d),
            "wn2": p["Wn2"].astype(BF16),
            "bn2": p["bn2"].reshape(1, d),
        }
        if with_coord:
            out["wc1"] = p["Wc1"].astype(BF16)
            out["bc1"] = p["bc1"].reshape(1, d)
            out["wc2t"] = p["Wc2"].reshape(1, d)
        return out

    nl = len(layers)
    ps = [split(p, i == nl - 1) for i, p in enumerate(layers)]

    x16 = jnp.pad(x, ((0, 0), (0, 16 - x.shape[1])))
    x128 = jnp.pad(x, ((0, 0), (0, 128 - x.shape[1])))
    zeros128 = jnp.zeros((NPAD, 128), F32)

    ea_n = 79872
    eb_n = e - ea_n
    bes = {ea_n: 1664, eb_n: 2504}
    halves = (
        (row[:ea_n], col[:ea_n], edge_attr[:ea_n]),
        (row[ea_n:], col[ea_n:], edge_attr[ea_n:]),
    )
    gd = {}
    g2 = {}
    seg = {}
    segp = {}
    for en in (ea_n, eb_n):
        gd[en] = _make_gather2sum(n, 128, en, True)
        g2[en] = _make_gather2(n, d // 2, en, jnp.int32)
        seg[en] = _make_segsum(en, d)
        segp[en] = _make_segsum_part(en)

    diffs = [gd[r.shape[0]](x128, x128, r, c) for r, c, _ in halves]

    hr, hc = _proj(h, ps[0]["wrlo"], ps[0]["wrhi"],
                   ps[0]["wclo"], ps[0]["wchi"])
    rads = [None, None]
    for i, p in enumerate(ps):
        gath = [g2[r.shape[0]](hr, hc, r, c) for r, c, _ in halves]
        m2s = []
        transs = []
        for j, (r, c, ea_h) in enumerate(halves):
            en = r.shape[0]
            if i == 0:
                m2, rads[j] = _edge_mlp(gath[j][0], gath[j][1], diffs[j],
                                        ea_h, p["wea"], p["wrad"], p["be1"],
                                        p["we2"], p["be2"], mode="first",
                                        be=bes[en])
            elif i < nl - 1:
                m2 = _edge_mlp(gath[j][0], gath[j][1], rads[j],
                               ea_h, p["wea"], p["wrad"], p["be1"],
                               p["we2"], p["be2"], mode="mid", be=bes[en])
            else:
                m2, trans = _edge_mlp(gath[j][0], gath[j][1], diffs[j],
                                      ea_h, p["wea"], p["wrad"], p["be1"],
                                      p["we2"], p["be2"], mode="last",
                                      coord=(p["wc1"], p["bc1"], p["wc2t"]),
                                      be=bes[en])
                transs.append(trans)
            m2s.append(m2)
        aggs = [seg[r.shape[0]](m2s[j], r, zeros128)
                for j, (r, c, _) in enumerate(halves)]
        if i < nl - 1:
            pn = ps[i + 1]
            h, hr, hc = _node_mlp(h, aggs[0], aggs[1],
                                  p["wn1h"], p["wn1a"], p["bn1"],
                                  p["wn2"], p["bn2"],
                                  pn["wrlo"], pn["wrhi"],
                                  pn["wclo"], pn["wchi"])
        else:
            parts = [segp[r.shape[0]](transs[j], r, zeros128)
                     for j, (r, c, _) in enumerate(halves)]
            pl4 = [parts[0][0, :n, :16], parts[0][1, :n, :16],
                   parts[1][0, :n, :16], parts[1][1, :n, :16]]
            h, x16o = _node_mlp_last(h, aggs[0], aggs[1], x16, pl4,
                                     p["wn1h"], p["wn1a"], p["bn1"],
                                     p["wn2"], p["bn2"])
    return (h, x16o[:, :3])


# speedup vs baseline: 4.1416x; 1.0010x over previous
import functools

import jax
import jax.numpy as jnp
from jax import lax
from jax.experimental import pallas as pl
from jax.experimental.pallas import tpu as pltpu
from jax.experimental.pallas import tpu_sc as plsc

F32 = jnp.float32

NC = 2
NS = 16
NW = NC * NS
NPAD = 10240
RPS = NPAD // NS


@functools.cache
def _mesh():
    return plsc.VectorSubcoreMesh(
        core_axis_name="c", subcore_axis_name="s",
        num_cores=NC, num_subcores=NS,
    )


@functools.cache
def _make_gather2sum(n_rows, width, n_edges, subtract, dtype=F32):
    per = n_edges // NW
    assert per * NW == n_edges
    CH = 64
    n_full = per // CH
    tail = per - n_full * CH
    ngroups = n_full // 3
    assert n_full % 3 == 0 and tail % 8 == 0
    isbf = dtype == jnp.bfloat16
    wdt = jnp.int32 if isbf else F32
    nslice = width // 16

    scratch = [
        pltpu.VMEM((per,), jnp.int32),
        pltpu.VMEM((per,), jnp.int32),
    ]
    for _ in range(3):
        scratch += [pltpu.VMEM((CH, width), wdt)] * 3
    scratch += [pltpu.SemaphoreType.DMA] * 6
    if tail:
        scratch += [pltpu.VMEM((tail, width), wdt)] * 3

    @functools.partial(
        pl.kernel,
        out_type=jax.ShapeDtypeStruct((n_edges, width), wdt),
        mesh=_mesh(),
        scratch_types=scratch,
    )
    def gather2sum(ta, tb, ia, ib, out, idxa, idxb,
                   a0, b0, o0, a1, b1, o1, a2, b2, o2,
                   sa0, sb0, sa1, sb1, sa2, sb2, *tails):
        wid = lax.axis_index("s") * NC + lax.axis_index("c")
        base0 = wid * per
        pltpu.sync_copy(ia.at[pl.ds(base0, per)], idxa)
        pltpu.sync_copy(ib.at[pl.ds(base0, per)], idxb)
        pairs = ((a0, b0, o0, sa0, sb0), (a1, b1, o1, sa1, sb1),
                 (a2, b2, o2, sa2, sb2))

        def issue(k, p):
            ba, bb, _, sa, sb = p
            pltpu.async_copy(ta.at[idxa.at[pl.ds(k * CH, CH)]], ba, sa)
            pltpu.async_copy(tb.at[idxb.at[pl.ds(k * CH, CH)]], bb, sb)

        def wait(k, p):
            ba, bb, _, sa, sb = p
            pltpu.make_async_copy(
                ta.at[idxa.at[pl.ds(k * CH, CH)]], ba, sa).wait()
            pltpu.make_async_copy(
                tb.at[idxb.at[pl.ds(k * CH, CH)]], bb, sb).wait()

        def combine(ba, bb, bo, n):
            def rowbody(r, c):
                for j in range(nslice):
                    sl = (r, pl.ds(j * 16, 16))
                    if isbf:
                        va = plsc.bitcast(ba[sl], jnp.bfloat16)
                        vb = plsc.bitcast(bb[sl], jnp.bfloat16)
                        v = va - vb if subtract else va + vb
                        bo[sl] = plsc.bitcast(v, jnp.int32)
                    else:
                        v = ba[sl] - bb[sl] if subtract else ba[sl] + bb[sl]
                        bo[sl] = v
                return c
            lax.fori_loop(0, n, rowbody, 0)

        for b in range(3):
            issue(b, pairs[b])

        def group(g, c):
            for b in range(3):
                k = 3 * g + b
                p = pairs[b]
                wait(k, p)
                combine(p[0], p[1], p[2], CH)
                issue(k + 3, p)
                pltpu.sync_copy(p[2], out.at[pl.ds(base0 + k * CH, CH)])
            return c

        lax.fori_loop(0, ngroups - 1, group, 0)
        for b in range(3):
            k = 3 * (ngroups - 1) + b
            p = pairs[b]
            wait(k, p)
            combine(p[0], p[1], p[2], CH)
            pltpu.sync_copy(p[2], out.at[pl.ds(base0 + k * CH, CH)])
        if tail:
            tba, tbb, tbo = tails
            kb = n_full * CH
            pltpu.async_copy(
                ta.at[idxa.at[pl.ds(kb, tail)]], tba, sa0).wait()
            pltpu.async_copy(
                tb.at[idxb.at[pl.ds(kb, tail)]], tbb, sb0).wait()
            combine(tba, tbb, tbo, tail)
            pltpu.sync_copy(tbo, out.at[pl.ds(base0 + kb, tail)])

    return gather2sum


@functools.cache
def _make_gather2(n_rows, width, n_edges, wdt):
    per = n_edges // NW
    assert per * NW == n_edges
    CH = next(c for c in range(104, 7, -8)
              if (per // c) % 3 == 0 and per // c >= 3)
    n_full = per // CH
    tail = per - n_full * CH
    ngroups = n_full // 3
    assert tail % 8 == 0
    scratch = [
        pltpu.VMEM((per,), jnp.int32),
        pltpu.VMEM((per,), jnp.int32),
    ]
    for _ in range(3):
        scratch += [pltpu.VMEM((CH, width), wdt)] * 2
    scratch += [pltpu.SemaphoreType.DMA] * 6
    if tail:
        scratch += [pltpu.VMEM((tail, width), wdt)] * 2

    @functools.partial(
        pl.kernel,
        out_type=(
            jax.ShapeDtypeStruct((n_edges, width), wdt),
            jax.ShapeDtypeStruct((n_edges, width), wdt),
        ),
        mesh=_mesh(),
        scratch_types=scratch,
    )
    def gather2(ta, tb, ia, ib, oa, ob, idxa, idxb,
                a0, b0, a1, b1, a2, b2,
                sa0, sb0, sa1, sb1, sa2, sb2, *tails):
        wid = lax.axis_index("s") * NC + lax.axis_index("c")
        base0 = wid * per
        pltpu.sync_copy(ia.at[pl.ds(base0, per)], idxa)
        pltpu.sync_copy(ib.at[pl.ds(base0, per)], idxb)
        pairs = ((a0, b0, sa0, sb0), (a1, b1, sa1, sb1), (a2, b2, sa2, sb2))

        def issue(k, p):
            ba, bb, sa, sb = p
            pltpu.async_copy(ta.at[idxa.at[pl.ds(k * CH, CH)]], ba, sa)
            pltpu.async_copy(tb.at[idxb.at[pl.ds(k * CH, CH)]], bb, sb)

        def wait(k, p):
            ba, bb, sa, sb = p
            pltpu.make_async_copy(
                ta.at[idxa.at[pl.ds(k * CH, CH)]], ba, sa).wait()
            pltpu.make_async_copy(
                tb.at[idxb.at[pl.ds(k * CH, CH)]], bb, sb).wait()

        def drain(k, p):
            ba, bb, sa, sb = p
            pltpu.sync_copy(ba, oa.at[pl.ds(base0 + k * CH, CH)])
            pltpu.sync_copy(bb, ob.at[pl.ds(base0 + k * CH, CH)])

        for b in range(3):
            issue(b, pairs[b])

        def group(g, c):
            for b in range(3):
                k = 3 * g + b
                p = pairs[b]
                wait(k, p)
                drain(k, p)
                issue(k + 3, p)
            return c

        lax.fori_loop(0, ngroups - 1, group, 0)
        for b in range(3):
            k = 3 * (ngroups - 1) + b
            p = pairs[b]
            wait(k, p)
            drain(k, p)
        if tail:
            tba, tbb = tails
            kb = n_full * CH
            pltpu.async_copy(
                ta.at[idxa.at[pl.ds(kb, tail)]], tba, sa0).wait()
            pltpu.async_copy(
                tb.at[idxb.at[pl.ds(kb, tail)]], tbb, sb0).wait()
            pltpu.sync_copy(tba, oa.at[pl.ds(base0 + kb, tail)])
            pltpu.sync_copy(tbb, ob.at[pl.ds(base0 + kb, tail)])

    return gather2


@functools.cache
def _make_segsum(n_edges, width):
    half = width // NC
    per = n_edges // NS
    assert per * NS == n_edges
    CH = 104
    n_full = per // CH
    tail = per - n_full * CH
    ngroups = n_full // 3
    assert n_full % 3 == 0 and tail % 8 == 0

    scratch = [
        pltpu.VMEM((CH,), jnp.int32),
        pltpu.VMEM((CH, half), F32),
        pltpu.VMEM((CH,), jnp.int32),
        pltpu.VMEM((CH, half), F32),
        pltpu.VMEM((CH,), jnp.int32),
        pltpu.VMEM((CH, half), F32),
        pltpu.VMEM_SHARED((NPAD, half), F32),
    ] + [pltpu.SemaphoreType.DMA] * 6
    if tail:
        scratch += [
            pltpu.VMEM((tail,), jnp.int32),
            pltpu.VMEM((tail, half), F32),
        ]

    @functools.partial(
        pl.kernel,
        out_type=jax.ShapeDtypeStruct((NPAD, width), F32),
        mesh=_mesh(),
        scratch_types=scratch,
    )
    def segsum(vals, rows, zeros, out, i0, v0, i1, v1, i2, v2, acc,
               si0, sv0, si1, sv1, si2, sv2, *tails):
        cid = lax.axis_index("c")
        sid = lax.axis_index("s")
        pltpu.sync_copy(
            zeros.at[pl.ds(sid * RPS, RPS), pl.ds(0, half)],
            acc.at[pl.ds(sid * RPS, RPS)],
        )
        plsc.subcore_barrier()

        base0 = sid * per
        pairs = ((i0, v0, si0, sv0), (i1, v1, si1, sv1), (i2, v2, si2, sv2))

        def issue(k, p):
            iv, vv, si, sv = p
            pltpu.async_copy(rows.at[pl.ds(base0 + k * CH, CH)], iv, si)
            pltpu.async_copy(
                vals.at[pl.ds(base0 + k * CH, CH), pl.ds(cid * half, half)],
                vv, sv)

        def wait(k, p):
            iv, vv, si, sv = p
            pltpu.make_async_copy(
                rows.at[pl.ds(base0 + k * CH, CH)], iv, si).wait()
            pltpu.make_async_copy(
                vals.at[pl.ds(base0 + k * CH, CH), pl.ds(cid * half, half)],
                vv, sv).wait()

        for b in range(3):
            issue(b, pairs[b])

        def group(g, c):
            for b in range(3):
                k = 3 * g + b
                p = pairs[b]
                wait(k, p)
                pltpu.sync_copy(p[1], acc.at[p[0]], add=True)
                issue(k + 3, p)
            return c

        lax.fori_loop(0, ngroups - 1, group, 0)
        for b in range(3):
            k = 3 * (ngroups - 1) + b
            p = pairs[b]
            wait(k, p)
            pltpu.sync_copy(p[1], acc.at[p[0]], add=True)
        if tail:
            ti, tv = tails
            kb = base0 + n_full * CH
            pltpu.sync_copy(rows.at[pl.ds(kb, tail)], ti)
            pltpu.sync_copy(
                vals.at[pl.ds(kb, tail), pl.ds(cid * half, half)], tv)
            pltpu.sync_copy(tv, acc.at[ti], add=True)

        plsc.subcore_barrier()
        pltpu.sync_copy(
            acc.at[pl.ds(sid * RPS, RPS)],
            out.at[pl.ds(sid * RPS, RPS), pl.ds(cid * half, half)],
        )

    return segsum


@functools.cache
def _make_segsum_part(n_edges):
    width = 128
    per_core = n_edges // NC
    per = per_core // NS
    CH = 64
    n_full = per // CH
    tail = per - n_full * CH
    ngroups = n_full // 3
    assert n_full % 3 == 0 and tail % 8 == 0

    scratch = [
        pltpu.VMEM((CH,), jnp.int32),
        pltpu.VMEM((CH, width), F32),
        pltpu.VMEM((CH,), jnp.int32),
        pltpu.VMEM((CH, width), F32),
        pltpu.VMEM((CH,), jnp.int32),
        pltpu.VMEM((CH, width), F32),
        pltpu.VMEM_SHARED((NPAD, width), F32),
    ] + [pltpu.SemaphoreType.DMA] * 6
    if tail:
        scratch += [
            pltpu.VMEM((tail,), jnp.int32),
            pltpu.VMEM((tail, width), F32),
        ]

    @functools.partial(
        pl.kernel,
        out_type=jax.ShapeDtypeStruct((NC, NPAD, width), F32),
        mesh=_mesh(),
        scratch_types=scratch,
    )
    def segsum_part(vals, rows, zeros, out, i0, v0, i1, v1, i2, v2, acc,
                    si0, sv0, si1, sv1, si2, sv2, *tails):
        cid = lax.axis_index("c")
        sid = lax.axis_index("s")
        pltpu.sync_copy(
            zeros.at[pl.ds(sid * RPS, RPS)],
            acc.at[pl.ds(sid * RPS, RPS)],
        )
        plsc.subcore_barrier()

        base0 = cid * per_core + sid * per
        pairs = ((i0, v0, si0, sv0), (i1, v1, si1, sv1), (i2, v2, si2, sv2))

        def issue(k, p):
            iv, vv, si, sv = p
            pltpu.async_copy(rows.at[pl.ds(base0 + k * CH, CH)], iv, si)
            pltpu.async_copy(vals.at[pl.ds(base0 + k * CH, CH)], vv, sv)

        def wait(k, p):
            iv, vv, si, sv = p
            pltpu.make_async_copy(
                rows.at[pl.ds(base0 + k * CH, CH)], iv, si).wait()
            pltpu.make_async_copy(
                vals.at[pl.ds(base0 + k * CH, CH)], vv, sv).wait()

        for b in range(3):
            issue(b, pairs[b])

        def group(g, c):
            for b in range(3):
                k = 3 * g + b
                p = pairs[b]
                wait(k, p)
                pltpu.sync_copy(p[1], acc.at[p[0]], add=True)
                issue(k + 3, p)
            return c

        lax.fori_loop(0, ngroups - 1, group, 0)
        for b in range(3):
            k = 3 * (ngroups - 1) + b
            p = pairs[b]
            wait(k, p)
            pltpu.sync_copy(p[1], acc.at[p[0]], add=True)
        if tail:
            ti, tv = tails
            kb = base0 + n_full * CH
            pltpu.sync_copy(rows.at[pl.ds(kb, tail)], ti)
            pltpu.sync_copy(vals.at[pl.ds(kb, tail)], tv)
            pltpu.sync_copy(tv, acc.at[ti], add=True)

        plsc.subcore_barrier()
        pltpu.sync_copy(
            acc.at[pl.ds(sid * RPS, RPS)],
            out.at[cid, pl.ds(sid * RPS, RPS)],
        )

    return segsum_part


def _silu(v):
    return v * (0.5 * jnp.tanh(v * 0.5) + 0.5)


def _dot(a, b):
    return jnp.dot(a, b, preferred_element_type=F32)


BF16 = jnp.bfloat16


def _dotb(a, b):
    return jnp.dot(a.astype(BF16), b, preferred_element_type=F32)


def _bf16_word(lo_f32, hi_f32):
    ulo = lax.bitcast_convert_type(lo_f32, jnp.uint32)
    uhi = lax.bitcast_convert_type(hi_f32, jnp.uint32)
    rlo = (ulo + 0x7FFF + ((ulo >> 16) & 1)) >> 16
    rhi = (uhi + 0x7FFF + ((uhi >> 16) & 1)) >> 16
    return lax.bitcast_convert_type(rlo | (rhi << 16), jnp.int32)


def _unpack_words(w32):
    u = lax.bitcast_convert_type(w32, jnp.uint32)
    lo = lax.bitcast_convert_type(u << 16, F32)
    hi = lax.bitcast_convert_type(u & jnp.uint32(0xFFFF0000), F32)
    return lo, hi


_BN = 2000
_BE = 1600


def _full(shape):
    return pl.BlockSpec(shape, lambda i: (0,) * len(shape))


def _proj_body(h, wrlo, wrhi, wclo, wchi, hr, hc):
    hv = h[...]
    hr[...] = _bf16_word(_dotb(hv, wrlo[...]), _dotb(hv, wrhi[...]))
    hc[...] = _bf16_word(_dotb(hv, wclo[...]), _dotb(hv, wchi[...]))


def _proj(h, wrlo, wrhi, wclo, wchi):
    n, d = h.shape
    hw = d // 2
    return pl.pallas_call(
        _proj_body,
        grid=(n // _BN,),
        in_specs=[
            pl.BlockSpec((_BN, d), lambda i: (i, 0)),
            _full((d, hw)), _full((d, hw)),
            _full((d, hw)), _full((d, hw)),
        ],
        out_specs=[pl.BlockSpec((_BN, hw), lambda i: (i, 0))] * 2,
        out_shape=[jax.ShapeDtypeStruct((n, hw), jnp.int32)] * 2,
    )(h, wrlo, wrhi, wclo, wchi)


def _edge_first_body(gr, gc, diff, ea, wea, wrad, be1, we2, be2,
                     out, rad8):
    d = diff[...][:, :16]
    radial = jnp.sum(d * d, axis=1, keepdims=True)
    rad8[...] = jnp.broadcast_to(radial, (radial.shape[0], 8))
    glo, ghi = _unpack_words(gr[...])
    clo, chi = _unpack_words(gc[...])
    msum = jnp.concatenate([glo + clo, ghi + chi], axis=1)
    pre = (
        msum + _dot(ea[...], wea[...])
        + radial * wrad[...] + be1[...]
    )
    m = _silu(pre)
    out[...] = _silu(_dotb(m, we2[...]) + be2[...])


def _edge_mid_body(gr, gc, rad, ea, wea, wrad, be1, we2, be2, out):
    radial = rad[...][:, :1]
    glo, ghi = _unpack_words(gr[...])
    clo, chi = _unpack_words(gc[...])
    msum = jnp.concatenate([glo + clo, ghi + chi], axis=1)
    pre = (
        msum + _dot(ea[...], wea[...])
        + radial * wrad[...] + be1[...]
    )
    m = _silu(pre)
    out[...] = _silu(_dotb(m, we2[...]) + be2[...])


def _edge_last_body(gr, gc, diff, ea, wea, wrad, be1, we2, be2,
                    wc1, bc1, wc2t, out, trans):
    d = diff[...][:, :16]
    radial = jnp.sum(d * d, axis=1, keepdims=True)
    glo, ghi = _unpack_words(gr[...])
    clo, chi = _unpack_words(gc[...])
    msum = jnp.concatenate([glo + clo, ghi + chi], axis=1)
    pre = (
        msum + _dot(ea[...], wea[...])
        + radial * wrad[...] + be1[...]
    )
    m = _silu(pre)
    m2 = _silu(_dotb(m, we2[...]) + be2[...])
    out[...] = m2
    c1 = _silu(_dotb(m2, wc1[...]) + bc1[...])
    w = jnp.sum(c1 * wc2t[...], axis=1, keepdims=True)
    trans[...] = jnp.concatenate(
        [d * w, jnp.zeros((d.shape[0], 112), F32)], axis=1
    )


def _edge_mlp(gr, gc, aux, ea, wea, wrad, be1, we2, be2,
              mode="mid", coord=None, be=_BE):
    e = gr.shape[0]
    d = gr.shape[1] * 2
    de = ea.shape[1]
    edge_spec = pl.BlockSpec((be, d), lambda i: (i, 0))
    word_spec = pl.BlockSpec((be, d // 2), lambda i: (i, 0))
    diff_spec = pl.BlockSpec((be, 128), lambda i: (i, 0))
    rad_spec = pl.BlockSpec((be, 8), lambda i: (i, 0))
    ea_spec = pl.BlockSpec((be, de), lambda i: (i, 0))
    aux_spec = rad_spec if mode == "mid" else diff_spec
    in_specs = [
        word_spec, word_spec, aux_spec, ea_spec,
        _full((de, d)), _full((1, d)), _full((1, d)),
        _full((d, d)), _full((1, d)),
    ]
    args = [gr, gc, aux, ea, wea, wrad, be1, we2, be2]
    if mode == "first":
        return pl.pallas_call(
            _edge_first_body,
            grid=(e // be,),
            in_specs=in_specs,
            out_specs=[edge_spec, rad_spec],
            out_shape=[
                jax.ShapeDtypeStruct((e, d), F32),
                jax.ShapeDtypeStruct((e, 8), F32),
            ],
        )(*args)
    if mode == "mid":
        return pl.pallas_call(
            _edge_mid_body,
            grid=(e // be,),
            in_specs=in_specs,
            out_specs=edge_spec,
            out_shape=jax.ShapeDtypeStruct((e, d), F32),
        )(*args)
    wc1, bc1, wc2t = coord
    return pl.pallas_call(
        _edge_last_body,
        grid=(e // be,),
        in_specs=in_specs + [_full((d, d)), _full((1, d)), _full((1, d))],
        out_specs=[edge_spec, pl.BlockSpec((be, 128), lambda i: (i, 0))],
        out_shape=[
            jax.ShapeDtypeStruct((e, d), F32),
            jax.ShapeDtypeStruct((e, 128), F32),
        ],
    )(*args, wc1, bc1, wc2t)


def _node_body(h, agga, aggb, wn1h, wn1a, bn1, wn2, bn2,
               wrlo, wrhi, wclo, wchi, out_h, out_hr, out_hc):
    hv = h[...]
    agg = agga[...] + aggb[...]
    t = _silu(_dotb(hv, wn1h[...]) + _dotb(agg, wn1a[...]) + bn1[...])
    hn = hv + _dotb(t, wn2[...]) + bn2[...]
    out_h[...] = hn
    out_hr[...] = _bf16_word(_dotb(hn, wrlo[...]), _dotb(hn, wrhi[...]))
    out_hc[...] = _bf16_word(_dotb(hn, wclo[...]), _dotb(hn, wchi[...]))


def _node_mlp(h, agga, aggb, wn1h, wn1a, bn1, wn2, bn2,
              wrlo, wrhi, wclo, wchi):
    n, d = h.shape
    hw = d // 2
    node_spec = pl.BlockSpec((_BN, d), lambda i: (i, 0))
    word_spec = pl.BlockSpec((_BN, hw), lambda i: (i, 0))
    return pl.pallas_call(
        _node_body,
        grid=(n // _BN,),
        in_specs=[
            node_spec, node_spec, node_spec,
            _full((d, d)), _full((d, d)), _full((1, d)),
            _full((d, d)), _full((1, d)),
            _full((d, hw)), _full((d, hw)),
            _full((d, hw)), _full((d, hw)),
        ],
        out_specs=[node_spec, word_spec, word_spec],
        out_shape=[jax.ShapeDtypeStruct((n, d), F32),
                   jax.ShapeDtypeStruct((n, hw), jnp.int32),
                   jax.ShapeDtypeStruct((n, hw), jnp.int32)],
    )(h, agga, aggb, wn1h, wn1a, bn1, wn2, bn2,
      wrlo, wrhi, wclo, wchi)


def _node_last_body(h, agga, aggb, x16, p0, p1, p2, p3,
                    wn1h, wn1a, bn1, wn2, bn2, out_h, out_x):
    hv = h[...]
    agg = agga[...] + aggb[...]
    t = _silu(_dotb(hv, wn1h[...]) + _dotb(agg, wn1a[...]) + bn1[...])
    out_h[...] = hv + _dotb(t, wn2[...]) + bn2[...]
    out_x[...] = x16[...] + p0[...] + p1[...] + p2[...] + p3[...]


def _node_mlp_last(h, agga, aggb, x16, parts,
                   wn1h, wn1a, bn1, wn2, bn2):
    n, d = h.shape
    node_spec = pl.BlockSpec((_BN, d), lambda i: (i, 0))
    nar_spec = pl.BlockSpec((_BN, 16), lambda i: (i, 0))
    return pl.pallas_call(
        _node_last_body,
        grid=(n // _BN,),
        in_specs=[
            node_spec, node_spec, node_spec,
            nar_spec, nar_spec, nar_spec, nar_spec, nar_spec,
            _full((d, d)), _full((d, d)), _full((1, d)),
            _full((d, d)), _full((1, d)),
        ],
        out_specs=[node_spec, nar_spec],
        out_shape=[
            jax.ShapeDtypeStruct((n, d), F32),
            jax.ShapeDtypeStruct((n, 16), F32),
        ],
    )(h, agga, aggb, x16, *parts, wn1h, wn1a, bn1, wn2, bn2)


def kernel(h, x, edges, edge_attr, params):
    layers = params["layers"]
    n, d = h.shape
    e = edges.shape[1]
    de = edge_attr.shape[1]
    row = edges[0]
    col = edges[1]

    def split(p, with_coord):
        we1 = p["We1"]
        out = {
            "wrlo": we1[:d, :d // 2].astype(BF16),
            "wrhi": we1[:d, d // 2:].astype(BF16),
            "wclo": we1[d:2 * d, :d // 2].astype(BF16),
            "wchi": we1[d:2 * d, d // 2:].astype(BF16),
            "wrad": we1[2 * d:2 * d + 1],
            "wea": we1[2 * d + 1:],
            "be1": p["be1"].reshape(1, d),
            "we2": p["We2"].astype(BF16),
            "be2": p["be2"].reshape(1, d),
            "wn1h": p["Wn1"][:d].astype(BF16),
            "wn1a": p["Wn1"][d:].astype(BF16),
            "bn1": p["bn1"].reshape(1, d),
            "wn2": p["Wn2"].astype(BF16),
            "bn2": p["bn2"].reshape(1, d),
        }
        if with_coord:
            out["wc1"] = p["Wc1"].astype(BF16)
            out["bc1"] = p["bc1"].reshape(1, d)
            out["wc2t"] = p["Wc2"].reshape(1, d)
        return out

    nl = len(layers)
    ps = [split(p, i == nl - 1) for i, p in enumerate(layers)]

    x16 = jnp.pad(x, ((0, 0), (0, 16 - x.shape[1])))
    x128 = jnp.pad(x, ((0, 0), (0, 128 - x.shape[1])))
    zeros128 = jnp.zeros((NPAD, 128), F32)

    ea_n = 79872
    eb_n = e - ea_n
    bes = {ea_n: 1664, eb_n: 2504}
    halves = (
        (row[:ea_n], col[:ea_n], edge_attr[:ea_n]),
        (row[ea_n:], col[ea_n:], edge_attr[ea_n:]),
    )
    gd = {}
    g2 = {}
    seg = {}
    segp = {}
    for en in (ea_n, eb_n):
        gd[en] = _make_gather2sum(n, 128, en, True)
        g2[en] = _make_gather2(n, d // 2, en, jnp.int32)
        seg[en] = _make_segsum(en, d)
        segp[en] = _make_segsum_part(en)

    diffs = [gd[r.shape[0]](x128, x128, r, c) for r, c, _ in halves]

    hr, hc = _proj(h, ps[0]["wrlo"], ps[0]["wrhi"],
                   ps[0]["wclo"], ps[0]["wchi"])
    rads = [None, None]
    for i, p in enumerate(ps):
        gath = [g2[r.shape[0]](hr, hc, r, c) for r, c, _ in halves]
        m2s = []
        transs = []
        for j, (r, c, ea_h) in enumerate(halves):
            en = r.shape[0]
            if i == 0:
                m2, rads[j] = _edge_mlp(gath[j][0], gath[j][1], diffs[j],
                                        ea_h, p["wea"], p["wrad"], p["be1"],
                                        p["we2"], p["be2"], mode="first",
                                        be=bes[en])
            elif i < nl - 1:
                m2 = _edge_mlp(gath[j][0], gath[j][1], rads[j],
                               ea_h, p["wea"], p["wrad"], p["be1"],
                               p["we2"], p["be2"], mode="mid", be=bes[en])
            else:
                m2, trans = _edge_mlp(gath[j][0], gath[j][1], diffs[j],
                                      ea_h, p["wea"], p["wrad"], p["be1"],
                                      p["we2"], p["be2"], mode="last",
                                      coord=(p["wc1"], p["bc1"], p["wc2t"]),
                                      be=bes[en])
                transs.append(trans)
            m2s.append(m2)
        aggs = [seg[r.shape[0]](m2s[j], r, zeros128)
                for j, (r, c, _) in enumerate(halves)]
        if i < nl - 1:
            pn = ps[i + 1]
            h, hr, hc = _node_mlp(h, aggs[0], aggs[1],
                                  p["wn1h"], p["wn1a"], p["bn1"],
                                  p["wn2"], p["bn2"],
                                  pn["wrlo"], pn["wrhi"],
                                  pn["wclo"], pn["wchi"])
        else:
            parts = [segp[r.shape[0]](transs[j], r, zeros128)
                     for j, (r, c, _) in enumerate(halves)]
            pl4 = [parts[0][0, :n, :16], parts[0][1, :n, :16],
                   parts[1][0, :n, :16], parts[1][1, :n, :16]]
            h, x16o = _node_mlp_last(h, aggs[0], aggs[1], x16, pl4,
                                     p["wn1h"], p["wn1a"], p["bn1"],
                                     p["wn2"], p["bn2"])
    return (h, x16o[:, :3])
